# baseline (speedup 1.0000x reference)
import functools

import jax
import jax.numpy as jnp
from jax import lax
from jax.experimental import pallas as pl
from jax.experimental.pallas import tpu as pltpu

N = 10000
E = 320000
D = 128
NT = 32
NB = 320
NPAD = NT * NB


def _mish(v):
    return v * jnp.tanh(jax.nn.softplus(v))


def _node_linear_body(x_ref, wd_ref, ws_ref, b_ref, a_ref, bb_ref):
    xb = x_ref[...]
    a_ref[...] = jnp.dot(xb, wd_ref[...], preferred_element_type=jnp.float32, precision=lax.Precision.HIGHEST) + b_ref[...]
    bb_ref[...] = jnp.dot(xb, ws_ref[...], preferred_element_type=jnp.float32, precision=lax.Precision.HIGHEST)


def _tc_node_linear(x, wd, ws, b):
    n = x.shape[0]
    blk = 1000
    grid = n // blk
    return pl.pallas_call(
        _node_linear_body,
        grid=(grid,),
        in_specs=[
            pl.BlockSpec((blk, D), lambda i: (i, 0)),
            pl.BlockSpec((D, D), lambda i: (0, 0)),
            pl.BlockSpec((D, D), lambda i: (0, 0)),
            pl.BlockSpec((1, D), lambda i: (0, 0)),
        ],
        out_specs=[
            pl.BlockSpec((blk, D), lambda i: (i, 0)),
            pl.BlockSpec((blk, D), lambda i: (i, 0)),
        ],
        out_shape=[
            jax.ShapeDtypeStruct((n, D), jnp.float32),
            jax.ShapeDtypeStruct((n, D), jnp.float32),
        ],
    )(x, wd, ws, b.reshape(1, D))


def _mlp_body(p_ref, w2_ref, b2_ref, h_ref):
    m = _mish(p_ref[...])
    h_ref[...] = jnp.dot(m, w2_ref[...], preferred_element_type=jnp.float32, precision=lax.Precision.HIGHEST) + b2_ref[...]


def _tc_mlp(p, w2, b2):
    blk = 1280
    grid = E // blk
    return pl.pallas_call(
        _mlp_body,
        grid=(grid,),
        in_specs=[
            pl.BlockSpec((blk, D), lambda i: (i, 0)),
            pl.BlockSpec((D, D), lambda i: (0, 0)),
            pl.BlockSpec((1, D), lambda i: (0, 0)),
        ],
        out_specs=pl.BlockSpec((blk, D), lambda i: (i, 0)),
        out_shape=jax.ShapeDtypeStruct((E, D), jnp.float32),
    )(p, w2, b2.reshape(1, D))


def _bn_mish_linear_body(h_ref, g_ref, be_ref, wd_ref, ws_ref, b_ref, a_ref, bb_ref):
    h = h_ref[...]
    mean = jnp.mean(h, axis=0, keepdims=True)
    var = jnp.mean((h - mean) ** 2, axis=0, keepdims=True)
    hn = (h - mean) * lax.rsqrt(var + 1e-5) * g_ref[...] + be_ref[...]
    hm = _mish(hn)
    a_ref[...] = jnp.dot(hm, wd_ref[...], preferred_element_type=jnp.float32, precision=lax.Precision.HIGHEST) + b_ref[...]
    bb_ref[...] = jnp.dot(hm, ws_ref[...], preferred_element_type=jnp.float32, precision=lax.Precision.HIGHEST)


def _tc_bn_mish_linear(h, g, be, wd, ws, b):
    return pl.pallas_call(
        _bn_mish_linear_body,
        out_shape=[
            jax.ShapeDtypeStruct((N, D), jnp.float32),
            jax.ShapeDtypeStruct((N, D), jnp.float32),
        ],
    )(h, g.reshape(1, D), be.reshape(1, D), wd, ws, b.reshape(1, D))


def _bn_mish_body(h_ref, g_ref, be_ref, o_ref):
    h = h_ref[...]
    mean = jnp.mean(h, axis=0, keepdims=True)
    var = jnp.mean((h - mean) ** 2, axis=0, keepdims=True)
    hn = (h - mean) * lax.rsqrt(var + 1e-5) * g_ref[...] + be_ref[...]
    o_ref[...] = _mish(hn)


def _tc_bn_mish(h, g, be):
    return pl.pallas_call(
        _bn_mish_body,
        out_shape=jax.ShapeDtypeStruct((N, D), jnp.float32),
    )(h, g.reshape(1, D), be.reshape(1, D))


def _gather_add(a, b, src, dst):
    return a[dst] + b[src]


def _segment_max(h, dst):
    out = jax.ops.segment_max(h, dst, num_segments=N)
    return jnp.where(jnp.isfinite(out), out, 0.0)


def kernel(x, edge_index, edge_attr, W1a, b1a, W2a, b2a, g1, be1, W1b, b1b, W2b, b2b, g2, be2):
    src = edge_index[0]
    dst = edge_index[1]
    wd1 = W1a[:D] - W1a[D:]
    ws1 = W1a[D:]
    wd2 = W1b[:D] - W1b[D:]
    ws2 = W1b[D:]

    a1, b1 = _tc_node_linear(x, wd1, ws1, b1a)
    p1 = _gather_add(a1, b1, src, dst)
    h1 = _tc_mlp(p1, W2a, b2a)
    m1 = _segment_max(h1, dst)
    a2, b2 = _tc_bn_mish_linear(m1, g1, be1, wd2, ws2, b1b)
    p2 = _gather_add(a2, b2, src, dst)
    h2 = _tc_mlp(p2, W2b, b2b)
    m2 = _segment_max(h2, dst)
    out = _tc_bn_mish(m2, g2, be2)
    return (out, edge_index, edge_attr)


# speedup vs baseline: 1.5682x; 1.5682x over previous
import functools

import jax
import jax.numpy as jnp
from jax import lax
from jax.experimental import pallas as pl
from jax.experimental.pallas import tpu as pltpu
from jax.experimental.pallas import tpu_sc as plsc

N = 10000
E = 320000
D = 128
NT = 32
NB = 320
NPAD = NT * NB
FLUSH = 2048
CAP = E + FLUSH
CHS = 128
CHG = 128
NEG = float("-inf")

_sc_mesh = plsc.VectorSubcoreMesh(core_axis_name="c", subcore_axis_name="s")


def _wid():
    return lax.axis_index("s") * 2 + lax.axis_index("c")


def _mish(v):
    return v * jnp.tanh(jax.nn.softplus(v))


def _node_linear_body(x_ref, wd_ref, ws_ref, b_ref, a_ref, bb_ref):
    xb = x_ref[...]
    a_ref[...] = jnp.dot(xb, wd_ref[...], preferred_element_type=jnp.float32, precision=lax.Precision.HIGHEST) + b_ref[...]
    bb_ref[...] = jnp.dot(xb, ws_ref[...], preferred_element_type=jnp.float32, precision=lax.Precision.HIGHEST)


def _tc_node_linear(x, wd, ws, b):
    n = x.shape[0]
    blk = 1000
    grid = n // blk
    return pl.pallas_call(
        _node_linear_body,
        grid=(grid,),
        in_specs=[
            pl.BlockSpec((blk, D), lambda i: (i, 0)),
            pl.BlockSpec((D, D), lambda i: (0, 0)),
            pl.BlockSpec((D, D), lambda i: (0, 0)),
            pl.BlockSpec((1, D), lambda i: (0, 0)),
        ],
        out_specs=[
            pl.BlockSpec((blk, D), lambda i: (i, 0)),
            pl.BlockSpec((blk, D), lambda i: (i, 0)),
        ],
        out_shape=[
            jax.ShapeDtypeStruct((n, D), jnp.float32),
            jax.ShapeDtypeStruct((n, D), jnp.float32),
        ],
    )(x, wd, ws, b.reshape(1, D))


def _mlp_body(p_ref, w2_ref, b2_ref, h_ref):
    m = _mish(p_ref[...])
    h_ref[...] = jnp.dot(m, w2_ref[...], preferred_element_type=jnp.float32, precision=lax.Precision.HIGHEST) + b2_ref[...]


def _tc_mlp(p, w2, b2):
    blk = 1280
    grid = E // blk
    return pl.pallas_call(
        _mlp_body,
        grid=(grid,),
        in_specs=[
            pl.BlockSpec((blk, D), lambda i: (i, 0)),
            pl.BlockSpec((D, D), lambda i: (0, 0)),
            pl.BlockSpec((1, D), lambda i: (0, 0)),
        ],
        out_specs=pl.BlockSpec((blk, D), lambda i: (i, 0)),
        out_shape=jax.ShapeDtypeStruct((E, D), jnp.float32),
    )(p, w2, b2.reshape(1, D))


def _bn_mish_linear_body(h_ref, g_ref, be_ref, wd_ref, ws_ref, b_ref, a_ref, bb_ref):
    h = h_ref[...]
    mean = jnp.mean(h, axis=0, keepdims=True)
    var = jnp.mean((h - mean) ** 2, axis=0, keepdims=True)
    hn = (h - mean) * lax.rsqrt(var + 1e-5) * g_ref[...] + be_ref[...]
    hm = _mish(hn)
    a_ref[...] = jnp.dot(hm, wd_ref[...], preferred_element_type=jnp.float32, precision=lax.Precision.HIGHEST) + b_ref[...]
    bb_ref[...] = jnp.dot(hm, ws_ref[...], preferred_element_type=jnp.float32, precision=lax.Precision.HIGHEST)


def _tc_bn_mish_linear(h, g, be, wd, ws, b):
    return pl.pallas_call(
        _bn_mish_linear_body,
        out_shape=[
            jax.ShapeDtypeStruct((N, D), jnp.float32),
            jax.ShapeDtypeStruct((N, D), jnp.float32),
        ],
    )(h, g.reshape(1, D), be.reshape(1, D), wd, ws, b.reshape(1, D))


def _bn_mish_body(h_ref, g_ref, be_ref, o_ref):
    h = h_ref[...]
    mean = jnp.mean(h, axis=0, keepdims=True)
    var = jnp.mean((h - mean) ** 2, axis=0, keepdims=True)
    hn = (h - mean) * lax.rsqrt(var + 1e-5) * g_ref[...] + be_ref[...]
    o_ref[...] = _mish(hn)


def _tc_bn_mish(h, g, be):
    return pl.pallas_call(
        _bn_mish_body,
        out_shape=jax.ShapeDtypeStruct((N, D), jnp.float32),
    )(h, g.reshape(1, D), be.reshape(1, D))


def _bucket_body(dst_hbm, ids_hbm, dloc_hbm, cnt_hbm, dstbuf, idsbuf, dlbuf, cbuf):
    wid = _wid()
    lo = wid * NB
    zero = jnp.zeros((16,), jnp.int32)

    def zstep(i, carry):
        idsbuf[pl.ds(i * 16, 16)] = zero
        dlbuf[pl.ds(i * 16, 16)] = zero
        return carry

    lax.fori_loop(0, (FLUSH + 16) // 16, zstep, 0)
    iota16 = lax.iota(jnp.int32, 16)

    CH = 3200

    def chunk(c, carry):
        pltpu.sync_copy(dst_hbm.at[pl.ds(pl.multiple_of(c * CH, 8), CH)], dstbuf)

        def step(i, carry2):
            cur, written = carry2
            v = dstbuf[pl.ds(i * 16, 16)]
            m = (v >= lo) & (v < lo + NB)
            c16 = jnp.cumsum(m.astype(jnp.int32))
            pos = cur + c16 - 1
            plsc.store_scatter(idsbuf, [pos], iota16 + (c * CH + i * 16), mask=m)
            plsc.store_scatter(dlbuf, [pos], v - lo, mask=m)
            cur = cur + c16[15]
            do = cur >= FLUSH

            @pl.when(do)
            def _():
                pltpu.sync_copy(idsbuf.at[pl.ds(0, FLUSH)], ids_hbm.at[pl.ds(pl.multiple_of(wid * CAP + written, 8), FLUSH)])
                pltpu.sync_copy(dlbuf.at[pl.ds(0, FLUSH)], dloc_hbm.at[pl.ds(pl.multiple_of(wid * CAP + written, 8), FLUSH)])
                idsbuf[pl.ds(0, 16)] = idsbuf[pl.ds(FLUSH, 16)]
                dlbuf[pl.ds(0, 16)] = dlbuf[pl.ds(FLUSH, 16)]

            cur = lax.select(do, cur - FLUSH, cur)
            written = lax.select(do, written + FLUSH, written)
            return (cur, written)

        return lax.fori_loop(0, CH // 16, step, carry)

    cur, written = lax.fori_loop(0, E // CH, chunk, (0, 0))
    pltpu.sync_copy(idsbuf.at[pl.ds(0, FLUSH)], ids_hbm.at[pl.ds(pl.multiple_of(wid * CAP + written, 8), FLUSH)])
    pltpu.sync_copy(dlbuf.at[pl.ds(0, FLUSH)], dloc_hbm.at[pl.ds(pl.multiple_of(wid * CAP + written, 8), FLUSH)])
    cbuf[pl.ds(0, 16)] = jnp.zeros((16,), jnp.int32) + (written + cur)
    pltpu.sync_copy(cbuf.at[pl.ds(0, 8)], cnt_hbm.at[pl.ds(pl.multiple_of(wid * 8, 8), 8)])


def _sc_bucket(dst):
    f = functools.partial(
        pl.kernel,
        out_type=[
            jax.ShapeDtypeStruct((NT * CAP,), jnp.int32),
            jax.ShapeDtypeStruct((NT * CAP,), jnp.int32),
            jax.ShapeDtypeStruct((NT * 8,), jnp.int32),
        ],
        mesh=_sc_mesh,
        compiler_params=pltpu.CompilerParams(needs_layout_passes=False),
        scratch_types=[
            pltpu.VMEM((3200,), jnp.int32),
            pltpu.VMEM((FLUSH + 16,), jnp.int32),
            pltpu.VMEM((FLUSH + 16,), jnp.int32),
            pltpu.VMEM((16,), jnp.int32),
        ],
    )
    return f(_bucket_body)(dst)


def _smax_body(h_hbm, ids_hbm, dloc_hbm, cnt_hbm, out_hbm, acc, idbuf, dlbuf, rows, cbuf, sem):
    wid = _wid()
    neg = jnp.full((16,), NEG, jnp.float32)

    def initr(r, carry):
        for k in range(8):
            acc[r, pl.ds(k * 16, 16)] = neg
        return carry

    lax.fori_loop(0, NB + 1, initr, 0)
    pltpu.sync_copy(cnt_hbm.at[pl.ds(pl.multiple_of(wid * 8, 8), 8)], cbuf.at[pl.ds(0, 8)])
    cnt = cbuf[pl.ds(0, 16)][0]
    nch = lax.div(cnt + (CHS - 1), CHS)

    def chunk(c, carry):
        off = c * CHS
        pltpu.sync_copy(ids_hbm.at[pl.ds(pl.multiple_of(wid * CAP + off, 8), CHS)], idbuf)
        pltpu.sync_copy(dloc_hbm.at[pl.ds(pl.multiple_of(wid * CAP + off, 8), CHS)], dlbuf.at[pl.ds(0, CHS)])
        pltpu.async_copy(h_hbm.at[idbuf], rows, sem).wait()

        def edge(e, carry2):
            ld = lax.select((off + e) < cnt, dlbuf[pl.ds(e, 16)][0], NB)
            for k in range(8):
                s = pl.ds(k * 16, 16)
                acc[ld, s] = jnp.maximum(acc[ld, s], rows[e, s])
            return carry2

        lax.fori_loop(0, CHS, edge, 0)
        return carry

    lax.fori_loop(0, nch, chunk, 0)

    def finr(r, carry):
        for k in range(8):
            s = pl.ds(k * 16, 16)
            v = acc[r, s]
            acc[r, s] = jnp.where(v == NEG, 0.0, v)
        return carry

    lax.fori_loop(0, NB, finr, 0)
    pltpu.sync_copy(acc.at[pl.ds(0, NB)], out_hbm.at[pl.ds(wid * NB, NB)])


def _sc_scatter_max(h, ids, dloc, cnts):
    f = functools.partial(
        pl.kernel,
        out_type=jax.ShapeDtypeStruct((NPAD, D), jnp.float32),
        mesh=_sc_mesh,
        compiler_params=pltpu.CompilerParams(needs_layout_passes=False),
        scratch_types=[
            pltpu.VMEM((NB + 1, D), jnp.float32),
            pltpu.VMEM((CHS,), jnp.int32),
            pltpu.VMEM((CHS + 16,), jnp.int32),
            pltpu.VMEM((CHS, D), jnp.float32),
            pltpu.VMEM((16,), jnp.int32),
            pltpu.SemaphoreType.DMA,
        ],
    )
    return f(_smax_body)(h, ids, dloc, cnts)


def _gadd_body(a_hbm, b_hbm, src_hbm, dst_hbm, p_hbm, sbuf, dbuf, ra, rb, sem, sem2):
    wid = _wid()
    nchunk = E // CHG

    def it(k, carry):
        c = wid + k * NT

        @pl.when(c < nchunk)
        def _():
            off = c * CHG
            pltpu.sync_copy(dst_hbm.at[pl.ds(pl.multiple_of(off, 8), CHG)], dbuf)
            pltpu.sync_copy(src_hbm.at[pl.ds(pl.multiple_of(off, 8), CHG)], sbuf)
            pltpu.async_copy(a_hbm.at[dbuf], ra, sem).wait()
            pltpu.async_copy(b_hbm.at[sbuf], rb, sem2).wait()

            def addr(r, carry2):
                for k2 in range(8):
                    s = pl.ds(k2 * 16, 16)
                    ra[r, s] = ra[r, s] + rb[r, s]
                return carry2

            lax.fori_loop(0, CHG, addr, 0)
            pltpu.sync_copy(ra, p_hbm.at[pl.ds(off, CHG)])

        return carry

    lax.fori_loop(0, (nchunk + NT - 1) // NT, it, 0)


def _sc_gather_add(a, b, src, dst):
    f = functools.partial(
        pl.kernel,
        out_type=jax.ShapeDtypeStruct((E, D), jnp.float32),
        mesh=_sc_mesh,
        compiler_params=pltpu.CompilerParams(needs_layout_passes=False),
        scratch_types=[
            pltpu.VMEM((CHG,), jnp.int32),
            pltpu.VMEM((CHG,), jnp.int32),
            pltpu.VMEM((CHG, D), jnp.float32),
            pltpu.VMEM((CHG, D), jnp.float32),
            pltpu.SemaphoreType.DMA,
            pltpu.SemaphoreType.DMA,
        ],
    )
    return f(_gadd_body)(a, b, src, dst)


def _gather_add(a, b, src, dst):
    return a[dst] + b[src]


def _segment_max(h, dst):
    out = jax.ops.segment_max(h, dst, num_segments=N)
    return jnp.where(jnp.isfinite(out), out, 0.0)


def kernel(x, edge_index, edge_attr, W1a, b1a, W2a, b2a, g1, be1, W1b, b1b, W2b, b2b, g2, be2):
    src = edge_index[0]
    dst = edge_index[1]
    wd1 = W1a[:D] - W1a[D:]
    ws1 = W1a[D:]
    wd2 = W1b[:D] - W1b[D:]
    ws2 = W1b[D:]

    ids, dloc, cnts = _sc_bucket(dst)
    a1, b1 = _tc_node_linear(x, wd1, ws1, b1a)
    p1 = _sc_gather_add(a1, b1, src, dst)
    h1 = _tc_mlp(p1, W2a, b2a)
    m1 = _sc_scatter_max(h1, ids, dloc, cnts)[:N]
    a2, b2 = _tc_bn_mish_linear(m1, g1, be1, wd2, ws2, b1b)
    p2 = _sc_gather_add(a2, b2, src, dst)
    h2 = _tc_mlp(p2, W2b, b2b)
    m2 = _sc_scatter_max(h2, ids, dloc, cnts)[:N]
    out = _tc_bn_mish(m2, g2, be2)
    return (out, edge_index, edge_attr)


# speedup vs baseline: 2.0589x; 1.3129x over previous
import functools

import jax
import jax.numpy as jnp
from jax import lax
from jax.experimental import pallas as pl
from jax.experimental.pallas import tpu as pltpu
from jax.experimental.pallas import tpu_sc as plsc

N = 10000
E = 320000
D = 128
NT = 32
NB = 320
NPAD = NT * NB
FLUSH = 2048
CAP = E + FLUSH
CHS = 128
CHG = 160
NEG = float("-inf")

_sc_mesh = plsc.VectorSubcoreMesh(core_axis_name="c", subcore_axis_name="s")


def _wid():
    return lax.axis_index("s") * 2 + lax.axis_index("c")


def _mish(v):
    return v * jnp.tanh(jax.nn.softplus(v))


def _node_linear_body(x_ref, wd_ref, ws_ref, b_ref, a_ref, bb_ref):
    xb = x_ref[...]
    a_ref[...] = jnp.dot(xb, wd_ref[...], preferred_element_type=jnp.float32, precision=lax.Precision.HIGHEST) + b_ref[...]
    bb_ref[...] = jnp.dot(xb, ws_ref[...], preferred_element_type=jnp.float32, precision=lax.Precision.HIGHEST)


def _tc_node_linear(x, wd, ws, b):
    n = x.shape[0]
    blk = 1000
    grid = n // blk
    return pl.pallas_call(
        _node_linear_body,
        grid=(grid,),
        in_specs=[
            pl.BlockSpec((blk, D), lambda i: (i, 0)),
            pl.BlockSpec((D, D), lambda i: (0, 0)),
            pl.BlockSpec((D, D), lambda i: (0, 0)),
            pl.BlockSpec((1, D), lambda i: (0, 0)),
        ],
        out_specs=[
            pl.BlockSpec((blk, D), lambda i: (i, 0)),
            pl.BlockSpec((blk, D), lambda i: (i, 0)),
        ],
        out_shape=[
            jax.ShapeDtypeStruct((n, D), jnp.float32),
            jax.ShapeDtypeStruct((n, D), jnp.float32),
        ],
    )(x, wd, ws, b.reshape(1, D))


def _mlp_body(p_ref, w2_ref, b2_ref, h_ref):
    m = _mish(p_ref[...])
    h_ref[...] = jnp.dot(m, w2_ref[...], preferred_element_type=jnp.float32, precision=lax.Precision.HIGHEST) + b2_ref[...]


def _tc_mlp(p, w2, b2):
    blk = 1280
    grid = E // blk
    return pl.pallas_call(
        _mlp_body,
        grid=(grid,),
        in_specs=[
            pl.BlockSpec((blk, D), lambda i: (i, 0)),
            pl.BlockSpec((D, D), lambda i: (0, 0)),
            pl.BlockSpec((1, D), lambda i: (0, 0)),
        ],
        out_specs=pl.BlockSpec((blk, D), lambda i: (i, 0)),
        out_shape=jax.ShapeDtypeStruct((E, D), jnp.float32),
    )(p, w2, b2.reshape(1, D))


def _bn_mish_linear_body(h_ref, g_ref, be_ref, wd_ref, ws_ref, b_ref, a_ref, bb_ref):
    h = h_ref[...]
    mean = jnp.mean(h, axis=0, keepdims=True)
    var = jnp.mean((h - mean) ** 2, axis=0, keepdims=True)
    hn = (h - mean) * lax.rsqrt(var + 1e-5) * g_ref[...] + be_ref[...]
    hm = _mish(hn)
    a_ref[...] = jnp.dot(hm, wd_ref[...], preferred_element_type=jnp.float32, precision=lax.Precision.HIGHEST) + b_ref[...]
    bb_ref[...] = jnp.dot(hm, ws_ref[...], preferred_element_type=jnp.float32, precision=lax.Precision.HIGHEST)


def _tc_bn_mish_linear(h, g, be, wd, ws, b):
    return pl.pallas_call(
        _bn_mish_linear_body,
        out_shape=[
            jax.ShapeDtypeStruct((N, D), jnp.float32),
            jax.ShapeDtypeStruct((N, D), jnp.float32),
        ],
    )(h, g.reshape(1, D), be.reshape(1, D), wd, ws, b.reshape(1, D))


def _bn_mish_body(h_ref, g_ref, be_ref, o_ref):
    h = h_ref[...]
    mean = jnp.mean(h, axis=0, keepdims=True)
    var = jnp.mean((h - mean) ** 2, axis=0, keepdims=True)
    hn = (h - mean) * lax.rsqrt(var + 1e-5) * g_ref[...] + be_ref[...]
    o_ref[...] = _mish(hn)


def _tc_bn_mish(h, g, be):
    return pl.pallas_call(
        _bn_mish_body,
        out_shape=jax.ShapeDtypeStruct((N, D), jnp.float32),
    )(h, g.reshape(1, D), be.reshape(1, D))


def _bucket_body(dst_hbm, lst_hbm, cnt_hbm, dstbuf, pbuf, cbuf):
    wid = _wid()
    lo = wid * NB
    zero = jnp.full((16,), NB, jnp.int32)

    def zstep(i, carry):
        pbuf[pl.ds(i * 16, 16)] = zero
        return carry

    lax.fori_loop(0, (FLUSH + 16) // 16, zstep, 0)
    iota16 = lax.iota(jnp.int32, 16)

    CH = 3200

    def chunk(c, carry):
        pltpu.sync_copy(dst_hbm.at[pl.ds(pl.multiple_of(c * CH, 8), CH)], dstbuf)

        def step(i, carry2):
            cur, written = carry2
            v = dstbuf[pl.ds(i * 16, 16)]
            m = (v >= lo) & (v < lo + NB)
            c16 = jnp.cumsum(m.astype(jnp.int32))
            pos = cur + c16 - 1
            packed = (iota16 + (c * CH + i * 16)) * 512 + (v - lo)
            plsc.store_scatter(pbuf, [pos], packed, mask=m)
            cur = cur + c16[15]
            do = cur >= FLUSH

            @pl.when(do)
            def _():
                pltpu.sync_copy(pbuf.at[pl.ds(0, FLUSH)], lst_hbm.at[pl.ds(pl.multiple_of(wid * CAP + written, 8), FLUSH)])
                pbuf[pl.ds(0, 16)] = pbuf[pl.ds(FLUSH, 16)]

            cur = lax.select(do, cur - FLUSH, cur)
            written = lax.select(do, written + FLUSH, written)
            return (cur, written)

        return lax.fori_loop(0, CH // 16, step, carry)

    cur, written = lax.fori_loop(0, E // CH, chunk, (0, 0))
    pltpu.sync_copy(pbuf.at[pl.ds(0, FLUSH)], lst_hbm.at[pl.ds(pl.multiple_of(wid * CAP + written, 8), FLUSH)])
    cbuf[pl.ds(0, 16)] = jnp.zeros((16,), jnp.int32) + (written + cur)
    pltpu.sync_copy(cbuf.at[pl.ds(0, 8)], cnt_hbm.at[pl.ds(pl.multiple_of(wid * 8, 8), 8)])


def _sc_bucket(dst):
    f = functools.partial(
        pl.kernel,
        out_type=[
            jax.ShapeDtypeStruct((NT * CAP,), jnp.int32),
            jax.ShapeDtypeStruct((NT * 8,), jnp.int32),
        ],
        mesh=_sc_mesh,
        compiler_params=pltpu.CompilerParams(needs_layout_passes=False),
        scratch_types=[
            pltpu.VMEM((3200,), jnp.int32),
            pltpu.VMEM((FLUSH + 16,), jnp.int32),
            pltpu.VMEM((16,), jnp.int32),
        ],
    )
    return f(_bucket_body)(dst)


def _smax_body(h_hbm, lst_hbm, cnt_hbm, out_hbm, acc, lbuf, idbuf, dlbuf, rows, cbuf, sem0, sem1):
    wid = _wid()
    neg = jnp.full((16,), NEG, jnp.float32)
    iota16 = lax.iota(jnp.int32, 16)
    sems = (sem0, sem1)

    def initr(r, carry):
        for k in range(8):
            acc[r, pl.ds(k * 16, 16)] = neg
        return carry

    lax.fori_loop(0, NB + 1, initr, 0)
    pltpu.sync_copy(cnt_hbm.at[pl.ds(pl.multiple_of(wid * 8, 8), 8)], cbuf.at[pl.ds(0, 8)])
    cnt = cbuf[pl.ds(0, 16)][0]
    nch = lax.div(cnt + (CHS - 1), CHS)

    def start(c, b):
        off = c * CHS
        pltpu.sync_copy(lst_hbm.at[pl.ds(pl.multiple_of(wid * CAP + off, 8), CHS)], lbuf.at[pl.ds(b * CHS, CHS)])
        for j in range(CHS // 16):
            lv = lbuf[pl.ds(b * CHS + j * 16, 16)]
            dl = lv & 511
            valid = (off + j * 16 + iota16) < cnt
            idbuf[pl.ds(b * CHS + j * 16, 16)] = lax.shift_right_logical(lv, 9)
            dlbuf[pl.ds(b * (CHS + 16) + j * 16, 16)] = jnp.where(valid, dl, NB)
        pltpu.async_copy(h_hbm.at[idbuf.at[pl.ds(b * CHS, CHS)]], rows.at[b], sems[b])

    def finish(b):
        pltpu.make_async_copy(h_hbm.at[idbuf.at[pl.ds(b * CHS, CHS)]], rows.at[b], sems[b]).wait()

        def edge(e, carry2):
            ld = dlbuf[pl.ds(b * (CHS + 16) + e, 16)][0]
            for k in range(8):
                s = pl.ds(k * 16, 16)
                acc[ld, s] = jnp.maximum(acc[ld, s], rows[b, e, s])
            return carry2

        lax.fori_loop(0, CHS, edge, 0)

    @pl.when(0 < nch)
    def _():
        start(0, 0)

    @pl.when(1 < nch)
    def _():
        start(1, 1)

    def pair(p, carry):
        for b in range(2):
            c = 2 * p + b

            @pl.when(c < nch)
            def _():
                finish(b)

                @pl.when(c + 2 < nch)
                def _():
                    start(c + 2, b)

        return carry

    lax.fori_loop(0, lax.div(nch + 1, 2), pair, 0)

    def finr(r, carry):
        for k in range(8):
            s = pl.ds(k * 16, 16)
            v = acc[r, s]
            acc[r, s] = jnp.where(v == NEG, 0.0, v)
        return carry

    lax.fori_loop(0, NB, finr, 0)
    pltpu.sync_copy(acc.at[pl.ds(0, NB)], out_hbm.at[pl.ds(wid * NB, NB)])


def _sc_scatter_max(h, lst, cnts):
    f = functools.partial(
        pl.kernel,
        out_type=jax.ShapeDtypeStruct((NPAD, D), jnp.float32),
        mesh=_sc_mesh,
        compiler_params=pltpu.CompilerParams(needs_layout_passes=False),
        scratch_types=[
            pltpu.VMEM((NB + 1, D), jnp.float32),
            pltpu.VMEM((2 * CHS,), jnp.int32),
            pltpu.VMEM((2 * CHS,), jnp.int32),
            pltpu.VMEM((2 * (CHS + 16),), jnp.int32),
            pltpu.VMEM((2, CHS, D), jnp.float32),
            pltpu.VMEM((16,), jnp.int32),
            pltpu.SemaphoreType.DMA,
            pltpu.SemaphoreType.DMA,
        ],
    )
    return f(_smax_body)(h, lst, cnts)


def _gadd_body(a_hbm, b_hbm, src_hbm, dst_hbm, p_hbm, sbuf, dbuf, ra, rb, sem0, sem1):
    wid = _wid()
    nchunk = E // CHG
    sems = (sem0, sem1)
    HG = CHG // 2

    def gathers(b):
        yield a_hbm.at[dbuf.at[pl.ds(b * CHG, HG)]], ra.at[b, pl.ds(0, HG)]
        yield a_hbm.at[dbuf.at[pl.ds(b * CHG + HG, HG)]], ra.at[b, pl.ds(HG, HG)]
        yield b_hbm.at[sbuf.at[pl.ds(b * CHG, HG)]], rb.at[b, pl.ds(0, HG)]
        yield b_hbm.at[sbuf.at[pl.ds(b * CHG + HG, HG)]], rb.at[b, pl.ds(HG, HG)]

    def start(c, b):
        off = c * CHG
        pltpu.sync_copy(dst_hbm.at[pl.ds(pl.multiple_of(off, 8), CHG)], dbuf.at[pl.ds(b * CHG, CHG)])
        pltpu.sync_copy(src_hbm.at[pl.ds(pl.multiple_of(off, 8), CHG)], sbuf.at[pl.ds(b * CHG, CHG)])
        for s_ref, d_ref in gathers(b):
            pltpu.async_copy(s_ref, d_ref, sems[b])

    def finish(c, b):
        for s_ref, d_ref in gathers(b):
            pltpu.make_async_copy(s_ref, d_ref, sems[b]).wait()

        def addr(r, carry2):
            for k2 in range(8):
                s = pl.ds(k2 * 16, 16)
                ra[b, r, s] = ra[b, r, s] + rb[b, r, s]
            return carry2

        lax.fori_loop(0, CHG, addr, 0)
        pltpu.sync_copy(ra.at[b], p_hbm.at[pl.ds(c * CHG, CHG)])

    c0 = wid
    c1 = wid + NT

    @pl.when(c0 < nchunk)
    def _():
        start(c0, 0)

    @pl.when(c1 < nchunk)
    def _():
        start(c1, 1)

    def it(k, carry):
        for b in range(2):
            c = wid + (2 * k + b) * NT

            @pl.when(c < nchunk)
            def _():
                finish(c, b)

                @pl.when(c + 2 * NT < nchunk)
                def _():
                    start(c + 2 * NT, b)

        return carry

    nit = (nchunk // NT + 2) // 2
    lax.fori_loop(0, nit, it, 0)


def _sc_gather_add(a, b, src, dst):
    f = functools.partial(
        pl.kernel,
        out_type=jax.ShapeDtypeStruct((E, D), jnp.float32),
        mesh=_sc_mesh,
        compiler_params=pltpu.CompilerParams(needs_layout_passes=False),
        scratch_types=[
            pltpu.VMEM((2 * CHG,), jnp.int32),
            pltpu.VMEM((2 * CHG,), jnp.int32),
            pltpu.VMEM((2, CHG, D), jnp.float32),
            pltpu.VMEM((2, CHG, D), jnp.float32),
            pltpu.SemaphoreType.DMA,
            pltpu.SemaphoreType.DMA,
        ],
    )
    return f(_gadd_body)(a, b, src, dst)


def kernel(x, edge_index, edge_attr, W1a, b1a, W2a, b2a, g1, be1, W1b, b1b, W2b, b2b, g2, be2):
    src = edge_index[0]
    dst = edge_index[1]
    wd1 = W1a[:D] - W1a[D:]
    ws1 = W1a[D:]
    wd2 = W1b[:D] - W1b[D:]
    ws2 = W1b[D:]

    lst, cnts = _sc_bucket(dst)
    a1, b1 = _tc_node_linear(x, wd1, ws1, b1a)
    p1 = _sc_gather_add(a1, b1, src, dst)
    h1 = _tc_mlp(p1, W2a, b2a)
    m1 = _sc_scatter_max(h1, lst, cnts)[:N]
    a2, b2 = _tc_bn_mish_linear(m1, g1, be1, wd2, ws2, b1b)
    p2 = _sc_gather_add(a2, b2, src, dst)
    h2 = _tc_mlp(p2, W2b, b2b)
    m2 = _sc_scatter_max(h2, lst, cnts)[:N]
    out = _tc_bn_mish(m2, g2, be2)
    return (out, edge_index, edge_attr)


# speedup vs baseline: 3.2468x; 1.5770x over previous
import functools

import jax
import jax.numpy as jnp
from jax import lax
from jax.experimental import pallas as pl
from jax.experimental.pallas import tpu as pltpu
from jax.experimental.pallas import tpu_sc as plsc

N = 10000
E = 320000
D = 128
NT = 32
NB = 320
NPAD = NT * NB
FLUSH = 2048
CAP = E + FLUSH
CHS = 128
CHG = 160
NEG = float("-inf")

_sc_mesh = plsc.VectorSubcoreMesh(core_axis_name="c", subcore_axis_name="s")


def _wid():
    return lax.axis_index("s") * 2 + lax.axis_index("c")


def _mish(v):
    return v * jnp.tanh(jax.nn.softplus(v))


def _node_linear_body(x_ref, wd_ref, ws_ref, b_ref, a_ref, bb_ref):
    xb = x_ref[...]
    a_ref[...] = jnp.dot(xb, wd_ref[...], preferred_element_type=jnp.float32, precision=lax.Precision.HIGHEST) + b_ref[...]
    bb_ref[...] = jnp.dot(xb, ws_ref[...], preferred_element_type=jnp.float32, precision=lax.Precision.HIGHEST)


def _tc_node_linear(x, wd, ws, b):
    n = x.shape[0]
    blk = 1000
    grid = n // blk
    return pl.pallas_call(
        _node_linear_body,
        grid=(grid,),
        in_specs=[
            pl.BlockSpec((blk, D), lambda i: (i, 0)),
            pl.BlockSpec((D, D), lambda i: (0, 0)),
            pl.BlockSpec((D, D), lambda i: (0, 0)),
            pl.BlockSpec((1, D), lambda i: (0, 0)),
        ],
        out_specs=[
            pl.BlockSpec((blk, D), lambda i: (i, 0)),
            pl.BlockSpec((blk, D), lambda i: (i, 0)),
        ],
        out_shape=[
            jax.ShapeDtypeStruct((n, D), jnp.float32),
            jax.ShapeDtypeStruct((n, D), jnp.float32),
        ],
    )(x, wd, ws, b.reshape(1, D))


def _mlp_body(p_ref, w2_ref, b2_ref, h_ref):
    m = _mish(p_ref[...])
    h_ref[...] = jnp.dot(m, w2_ref[...], preferred_element_type=jnp.float32, precision=lax.Precision.HIGHEST) + b2_ref[...]


def _tc_mlp(p, w2, b2):
    blk = 1280
    grid = E // blk
    return pl.pallas_call(
        _mlp_body,
        grid=(grid,),
        in_specs=[
            pl.BlockSpec((blk, D), lambda i: (i, 0)),
            pl.BlockSpec((D, D), lambda i: (0, 0)),
            pl.BlockSpec((1, D), lambda i: (0, 0)),
        ],
        out_specs=pl.BlockSpec((blk, D), lambda i: (i, 0)),
        out_shape=jax.ShapeDtypeStruct((E, D), jnp.float32),
    )(p, w2, b2.reshape(1, D))


def _bn_mish_linear_body(h_ref, g_ref, be_ref, wd_ref, ws_ref, b_ref, a_ref, bb_ref):
    h = h_ref[...]
    mean = jnp.mean(h, axis=0, keepdims=True)
    var = jnp.mean((h - mean) ** 2, axis=0, keepdims=True)
    hn = (h - mean) * lax.rsqrt(var + 1e-5) * g_ref[...] + be_ref[...]
    hm = _mish(hn)
    a_ref[...] = jnp.dot(hm, wd_ref[...], preferred_element_type=jnp.float32, precision=lax.Precision.HIGHEST) + b_ref[...]
    bb_ref[...] = jnp.dot(hm, ws_ref[...], preferred_element_type=jnp.float32, precision=lax.Precision.HIGHEST)


def _tc_bn_mish_linear(h, g, be, wd, ws, b):
    return pl.pallas_call(
        _bn_mish_linear_body,
        out_shape=[
            jax.ShapeDtypeStruct((N, D), jnp.float32),
            jax.ShapeDtypeStruct((N, D), jnp.float32),
        ],
    )(h, g.reshape(1, D), be.reshape(1, D), wd, ws, b.reshape(1, D))


def _bn_mish_body(h_ref, g_ref, be_ref, o_ref):
    h = h_ref[...]
    mean = jnp.mean(h, axis=0, keepdims=True)
    var = jnp.mean((h - mean) ** 2, axis=0, keepdims=True)
    hn = (h - mean) * lax.rsqrt(var + 1e-5) * g_ref[...] + be_ref[...]
    o_ref[...] = _mish(hn)


def _tc_bn_mish(h, g, be):
    return pl.pallas_call(
        _bn_mish_body,
        out_shape=jax.ShapeDtypeStruct((N, D), jnp.float32),
    )(h, g.reshape(1, D), be.reshape(1, D))


def _bucket_body(dst_hbm, lst_hbm, cnt_hbm, dstbuf, pbuf, cbuf):
    wid = _wid()
    lo = wid * NB
    zero = jnp.full((16,), NB, jnp.int32)

    def zstep(i, carry):
        pbuf[pl.ds(i * 16, 16)] = zero
        return carry

    lax.fori_loop(0, (FLUSH + 80) // 16, zstep, 0)
    iota16 = lax.iota(jnp.int32, 16)

    CH = 3200

    def chunk(c, carry):
        pltpu.sync_copy(dst_hbm.at[pl.ds(pl.multiple_of(c * CH, 8), CH)], dstbuf)

        def step(i, carry2):
            cur, written = carry2
            base = c * CH + i * 64
            vs = [dstbuf[pl.ds(i * 64 + u * 16, 16)] for u in range(4)]
            ms = [(v >= lo) & (v < lo + NB) for v in vs]
            cs = [jnp.cumsum(m.astype(jnp.int32)) for m in ms]
            cnts = [cc[15] for cc in cs]
            packs = [(iota16 + (base + u * 16)) * 512 + (vs[u] - lo) for u in range(4)]
            for u in range(4):
                plsc.store_scatter(pbuf, [cur + cs[u] - 1], packs[u], mask=ms[u])
                cur = cur + cnts[u]
            do = cur >= FLUSH

            @pl.when(do)
            def _():
                pltpu.sync_copy(pbuf.at[pl.ds(0, FLUSH)], lst_hbm.at[pl.ds(pl.multiple_of(wid * CAP + written, 8), FLUSH)])
                for u in range(5):
                    pbuf[pl.ds(u * 16, 16)] = pbuf[pl.ds(FLUSH + u * 16, 16)]

            cur = lax.select(do, cur - FLUSH, cur)
            written = lax.select(do, written + FLUSH, written)
            return (cur, written)

        return lax.fori_loop(0, CH // 64, step, carry)

    cur, written = lax.fori_loop(0, E // CH, chunk, (0, 0))
    pltpu.sync_copy(pbuf.at[pl.ds(0, FLUSH)], lst_hbm.at[pl.ds(pl.multiple_of(wid * CAP + written, 8), FLUSH)])
    cbuf[pl.ds(0, 16)] = jnp.zeros((16,), jnp.int32) + (written + cur)
    pltpu.sync_copy(cbuf.at[pl.ds(0, 8)], cnt_hbm.at[pl.ds(pl.multiple_of(wid * 8, 8), 8)])


def _sc_bucket(dst):
    f = functools.partial(
        pl.kernel,
        out_type=[
            jax.ShapeDtypeStruct((NT * CAP,), jnp.int32),
            jax.ShapeDtypeStruct((NT * 8,), jnp.int32),
        ],
        mesh=_sc_mesh,
        compiler_params=pltpu.CompilerParams(needs_layout_passes=False),
        scratch_types=[
            pltpu.VMEM((3200,), jnp.int32),
            pltpu.VMEM((FLUSH + 80,), jnp.int32),
            pltpu.VMEM((16,), jnp.int32),
        ],
    )
    return f(_bucket_body)(dst)


def _smax_body(h_hbm, lst_hbm, cnt_hbm, out_hbm, acc, lbuf, idbuf, dlbuf, rows, cbuf, sem0, sem1):
    wid = _wid()
    neg = jnp.full((16,), NEG, jnp.float32)
    iota16 = lax.iota(jnp.int32, 16)
    sems = (sem0, sem1)

    def initr(r, carry):
        for k in range(8):
            acc[r, pl.ds(k * 16, 16)] = neg
        return carry

    lax.fori_loop(0, NB + 1, initr, 0)
    pltpu.sync_copy(cnt_hbm.at[pl.ds(pl.multiple_of(wid * 8, 8), 8)], cbuf.at[pl.ds(0, 8)])
    cnt = cbuf[pl.ds(0, 16)][0]
    nch = lax.div(cnt + (CHS - 1), CHS)

    def start(c, b):
        off = c * CHS
        pltpu.sync_copy(lst_hbm.at[pl.ds(pl.multiple_of(wid * CAP + off, 8), CHS)], lbuf.at[pl.ds(b * CHS, CHS)])
        for j in range(CHS // 16):
            lv = lbuf[pl.ds(b * CHS + j * 16, 16)]
            dl = lv & 511
            valid = (off + j * 16 + iota16) < cnt
            idbuf[pl.ds(b * CHS + j * 16, 16)] = lax.shift_right_logical(lv, 9)
            dlbuf[pl.ds(b * (CHS + 16) + j * 16, 16)] = jnp.where(valid, dl, NB)
        pltpu.async_copy(h_hbm.at[idbuf.at[pl.ds(b * CHS, CHS)]], rows.at[b], sems[b])

    def finish(b):
        pltpu.make_async_copy(h_hbm.at[idbuf.at[pl.ds(b * CHS, CHS)]], rows.at[b], sems[b]).wait()

        def edge(e2, carry2):
            e = e2 * 2
            ld0 = dlbuf[pl.ds(b * (CHS + 16) + e, 16)][0]
            ld1 = dlbuf[pl.ds(b * (CHS + 16) + e + 1, 16)][0]
            r0 = [rows[b, e, pl.ds(k * 16, 16)] for k in range(8)]
            r1 = [rows[b, e + 1, pl.ds(k * 16, 16)] for k in range(8)]
            for k in range(8):
                s = pl.ds(k * 16, 16)
                acc[ld0, s] = jnp.maximum(acc[ld0, s], r0[k])
            for k in range(8):
                s = pl.ds(k * 16, 16)
                acc[ld1, s] = jnp.maximum(acc[ld1, s], r1[k])
            return carry2

        lax.fori_loop(0, CHS // 2, edge, 0)

    @pl.when(0 < nch)
    def _():
        start(0, 0)

    @pl.when(1 < nch)
    def _():
        start(1, 1)

    def pair(p, carry):
        for b in range(2):
            c = 2 * p + b

            @pl.when(c < nch)
            def _():
                finish(b)

                @pl.when(c + 2 < nch)
                def _():
                    start(c + 2, b)

        return carry

    lax.fori_loop(0, lax.div(nch + 1, 2), pair, 0)

    def finr(r, carry):
        for k in range(8):
            s = pl.ds(k * 16, 16)
            v = acc[r, s]
            acc[r, s] = jnp.where(v == NEG, 0.0, v)
        return carry

    lax.fori_loop(0, NB, finr, 0)
    pltpu.sync_copy(acc.at[pl.ds(0, NB)], out_hbm.at[pl.ds(wid * NB, NB)])


def _sc_scatter_max(h, lst, cnts):
    f = functools.partial(
        pl.kernel,
        out_type=jax.ShapeDtypeStruct((NPAD, D), jnp.float32),
        mesh=_sc_mesh,
        compiler_params=pltpu.CompilerParams(needs_layout_passes=False),
        scratch_types=[
            pltpu.VMEM((NB + 1, D), jnp.float32),
            pltpu.VMEM((2 * CHS,), jnp.int32),
            pltpu.VMEM((2 * CHS,), jnp.int32),
            pltpu.VMEM((2 * (CHS + 16),), jnp.int32),
            pltpu.VMEM((2, CHS, D), jnp.float32),
            pltpu.VMEM((16,), jnp.int32),
            pltpu.SemaphoreType.DMA,
            pltpu.SemaphoreType.DMA,
        ],
    )
    return f(_smax_body)(h, lst, cnts)


def _gadd_body(a_hbm, b_hbm, src_hbm, dst_hbm, p_hbm, sbuf, dbuf, ra, rb, sem0, sem1):
    wid = _wid()
    nchunk = E // CHG
    sems = (sem0, sem1)
    HG = CHG // 2

    def gathers(b):
        yield a_hbm.at[dbuf.at[pl.ds(b * CHG, HG)]], ra.at[b, pl.ds(0, HG)]
        yield a_hbm.at[dbuf.at[pl.ds(b * CHG + HG, HG)]], ra.at[b, pl.ds(HG, HG)]
        yield b_hbm.at[sbuf.at[pl.ds(b * CHG, HG)]], rb.at[b, pl.ds(0, HG)]
        yield b_hbm.at[sbuf.at[pl.ds(b * CHG + HG, HG)]], rb.at[b, pl.ds(HG, HG)]

    def start(c, b):
        off = c * CHG
        pltpu.sync_copy(dst_hbm.at[pl.ds(pl.multiple_of(off, 8), CHG)], dbuf.at[pl.ds(b * CHG, CHG)])
        pltpu.sync_copy(src_hbm.at[pl.ds(pl.multiple_of(off, 8), CHG)], sbuf.at[pl.ds(b * CHG, CHG)])
        for s_ref, d_ref in gathers(b):
            pltpu.async_copy(s_ref, d_ref, sems[b])

    def finish(c, b):
        for s_ref, d_ref in gathers(b):
            pltpu.make_async_copy(s_ref, d_ref, sems[b]).wait()

        def addr(r, carry2):
            for k2 in range(8):
                s = pl.ds(k2 * 16, 16)
                ra[b, r, s] = ra[b, r, s] + rb[b, r, s]
            return carry2

        lax.fori_loop(0, CHG, addr, 0)
        pltpu.sync_copy(ra.at[b], p_hbm.at[pl.ds(c * CHG, CHG)])

    c0 = wid
    c1 = wid + NT

    @pl.when(c0 < nchunk)
    def _():
        start(c0, 0)

    @pl.when(c1 < nchunk)
    def _():
        start(c1, 1)

    def it(k, carry):
        for b in range(2):
            c = wid + (2 * k + b) * NT

            @pl.when(c < nchunk)
            def _():
                finish(c, b)

                @pl.when(c + 2 * NT < nchunk)
                def _():
                    start(c + 2 * NT, b)

        return carry

    nit = (nchunk // NT + 2) // 2
    lax.fori_loop(0, nit, it, 0)


def _sc_gather_add(a, b, src, dst):
    f = functools.partial(
        pl.kernel,
        out_type=jax.ShapeDtypeStruct((E, D), jnp.float32),
        mesh=_sc_mesh,
        compiler_params=pltpu.CompilerParams(needs_layout_passes=False),
        scratch_types=[
            pltpu.VMEM((2 * CHG,), jnp.int32),
            pltpu.VMEM((2 * CHG,), jnp.int32),
            pltpu.VMEM((2, CHG, D), jnp.float32),
            pltpu.VMEM((2, CHG, D), jnp.float32),
            pltpu.SemaphoreType.DMA,
            pltpu.SemaphoreType.DMA,
        ],
    )
    return f(_gadd_body)(a, b, src, dst)


def kernel(x, edge_index, edge_attr, W1a, b1a, W2a, b2a, g1, be1, W1b, b1b, W2b, b2b, g2, be2):
    src = edge_index[0]
    dst = edge_index[1]
    wd1 = W1a[:D] - W1a[D:]
    ws1 = W1a[D:]
    wd2 = W1b[:D] - W1b[D:]
    ws2 = W1b[D:]

    lst, cnts = _sc_bucket(dst)
    a1, b1 = _tc_node_linear(x, wd1, ws1, b1a)
    p1 = _sc_gather_add(a1, b1, src, dst)
    h1 = _tc_mlp(p1, W2a, b2a)
    m1 = _sc_scatter_max(h1, lst, cnts)[:N]
    a2, b2 = _tc_bn_mish_linear(m1, g1, be1, wd2, ws2, b1b)
    p2 = _sc_gather_add(a2, b2, src, dst)
    h2 = _tc_mlp(p2, W2b, b2b)
    m2 = _sc_scatter_max(h2, lst, cnts)[:N]
    out = _tc_bn_mish(m2, g2, be2)
    return (out, edge_index, edge_attr)


# speedup vs baseline: 3.2595x; 1.0039x over previous
import functools

import jax
import jax.numpy as jnp
from jax import lax
from jax.experimental import pallas as pl
from jax.experimental.pallas import tpu as pltpu
from jax.experimental.pallas import tpu_sc as plsc

N = 10000
E = 320000
D = 128
NT = 32
NB = 320
NPAD = NT * NB
FLUSH = 2048
CAP = E + FLUSH
CHS = 128
CHG = 160
NEG = float("-inf")

_sc_mesh = plsc.VectorSubcoreMesh(core_axis_name="c", subcore_axis_name="s")


def _wid():
    return lax.axis_index("s") * 2 + lax.axis_index("c")


def _mish(v):
    return v * jnp.tanh(jax.nn.softplus(v))


def _node_linear_body(x_ref, wd_ref, ws_ref, b_ref, a_ref, bb_ref):
    xb = x_ref[...]
    a_ref[...] = jnp.dot(xb, wd_ref[...], preferred_element_type=jnp.float32, precision=lax.Precision.HIGHEST) + b_ref[...]
    bb_ref[...] = jnp.dot(xb, ws_ref[...], preferred_element_type=jnp.float32, precision=lax.Precision.HIGHEST)


def _tc_node_linear(x, wd, ws, b):
    n = x.shape[0]
    blk = 1000
    grid = n // blk
    return pl.pallas_call(
        _node_linear_body,
        grid=(grid,),
        in_specs=[
            pl.BlockSpec((blk, D), lambda i: (i, 0)),
            pl.BlockSpec((D, D), lambda i: (0, 0)),
            pl.BlockSpec((D, D), lambda i: (0, 0)),
            pl.BlockSpec((1, D), lambda i: (0, 0)),
        ],
        out_specs=[
            pl.BlockSpec((blk, D), lambda i: (i, 0)),
            pl.BlockSpec((blk, D), lambda i: (i, 0)),
        ],
        out_shape=[
            jax.ShapeDtypeStruct((n, D), jnp.float32),
            jax.ShapeDtypeStruct((n, D), jnp.float32),
        ],
    )(x, wd, ws, b.reshape(1, D))


def _mlp_body(p_ref, w2_ref, b2_ref, h_ref):
    m = _mish(p_ref[...])
    h_ref[...] = jnp.dot(m, w2_ref[...], preferred_element_type=jnp.float32, precision=lax.Precision.HIGHEST) + b2_ref[...]


def _tc_mlp(p, w2, b2):
    blk = 1280
    grid = E // blk
    return pl.pallas_call(
        _mlp_body,
        grid=(grid,),
        in_specs=[
            pl.BlockSpec((blk, D), lambda i: (i, 0)),
            pl.BlockSpec((D, D), lambda i: (0, 0)),
            pl.BlockSpec((1, D), lambda i: (0, 0)),
        ],
        out_specs=pl.BlockSpec((blk, D), lambda i: (i, 0)),
        out_shape=jax.ShapeDtypeStruct((E, D), jnp.float32),
    )(p, w2, b2.reshape(1, D))


def _bn_mish_linear_body(h_ref, g_ref, be_ref, wd_ref, ws_ref, b_ref, a_ref, bb_ref):
    h = h_ref[...]
    mean = jnp.mean(h, axis=0, keepdims=True)
    var = jnp.mean((h - mean) ** 2, axis=0, keepdims=True)
    hn = (h - mean) * lax.rsqrt(var + 1e-5) * g_ref[...] + be_ref[...]
    hm = _mish(hn)
    a_ref[...] = jnp.dot(hm, wd_ref[...], preferred_element_type=jnp.float32, precision=lax.Precision.HIGHEST) + b_ref[...]
    bb_ref[...] = jnp.dot(hm, ws_ref[...], preferred_element_type=jnp.float32, precision=lax.Precision.HIGHEST)


def _tc_bn_mish_linear(h, g, be, wd, ws, b):
    return pl.pallas_call(
        _bn_mish_linear_body,
        out_shape=[
            jax.ShapeDtypeStruct((N, D), jnp.float32),
            jax.ShapeDtypeStruct((N, D), jnp.float32),
        ],
    )(h, g.reshape(1, D), be.reshape(1, D), wd, ws, b.reshape(1, D))


def _bn_mish_body(h_ref, g_ref, be_ref, o_ref):
    h = h_ref[...]
    mean = jnp.mean(h, axis=0, keepdims=True)
    var = jnp.mean((h - mean) ** 2, axis=0, keepdims=True)
    hn = (h - mean) * lax.rsqrt(var + 1e-5) * g_ref[...] + be_ref[...]
    o_ref[...] = _mish(hn)


def _tc_bn_mish(h, g, be):
    return pl.pallas_call(
        _bn_mish_body,
        out_shape=jax.ShapeDtypeStruct((N, D), jnp.float32),
    )(h, g.reshape(1, D), be.reshape(1, D))


def _bucket_body(dst_hbm, lst_hbm, cnt_hbm, dstbuf, pbuf, cbuf):
    wid = _wid()
    lo = wid * NB
    zero = jnp.full((16,), NB, jnp.int32)

    def zstep(i, carry):
        pbuf[pl.ds(i * 16, 16)] = zero
        return carry

    lax.fori_loop(0, (FLUSH + 144) // 16, zstep, 0)
    iota16 = lax.iota(jnp.int32, 16)

    CH = 3200

    def chunk(c, carry):
        pltpu.sync_copy(dst_hbm.at[pl.ds(pl.multiple_of(c * CH, 8), CH)], dstbuf)

        def step(i, carry2):
            cur, written = carry2
            base = c * CH + i * 128
            vs = [dstbuf[pl.ds(i * 128 + u * 16, 16)] for u in range(8)]
            ms = [(v >= lo) & (v < lo + NB) for v in vs]
            cs = [jnp.cumsum(m.astype(jnp.int32)) for m in ms]
            cnts = [cc[15] for cc in cs]
            packs = [(iota16 + (base + u * 16)) * 512 + (vs[u] - lo) for u in range(8)]
            for u in range(8):
                plsc.store_scatter(pbuf, [cur + cs[u] - 1], packs[u], mask=ms[u])
                cur = cur + cnts[u]
            do = cur >= FLUSH

            @pl.when(do)
            def _():
                pltpu.sync_copy(pbuf.at[pl.ds(0, FLUSH)], lst_hbm.at[pl.ds(pl.multiple_of(wid * CAP + written, 8), FLUSH)])
                for u in range(9):
                    pbuf[pl.ds(u * 16, 16)] = pbuf[pl.ds(FLUSH + u * 16, 16)]

            cur = lax.select(do, cur - FLUSH, cur)
            written = lax.select(do, written + FLUSH, written)
            return (cur, written)

        return lax.fori_loop(0, CH // 128, step, carry)

    cur, written = lax.fori_loop(0, E // CH, chunk, (0, 0))
    pltpu.sync_copy(pbuf.at[pl.ds(0, FLUSH)], lst_hbm.at[pl.ds(pl.multiple_of(wid * CAP + written, 8), FLUSH)])
    cbuf[pl.ds(0, 16)] = jnp.zeros((16,), jnp.int32) + (written + cur)
    pltpu.sync_copy(cbuf.at[pl.ds(0, 8)], cnt_hbm.at[pl.ds(pl.multiple_of(wid * 8, 8), 8)])


def _sc_bucket(dst):
    f = functools.partial(
        pl.kernel,
        out_type=[
            jax.ShapeDtypeStruct((NT * CAP,), jnp.int32),
            jax.ShapeDtypeStruct((NT * 8,), jnp.int32),
        ],
        mesh=_sc_mesh,
        compiler_params=pltpu.CompilerParams(needs_layout_passes=False),
        scratch_types=[
            pltpu.VMEM((3200,), jnp.int32),
            pltpu.VMEM((FLUSH + 144,), jnp.int32),
            pltpu.VMEM((16,), jnp.int32),
        ],
    )
    return f(_bucket_body)(dst)


def _smax_body(h_hbm, lst_hbm, cnt_hbm, out_hbm, acc, lbuf, idbuf, dlbuf, rows, cbuf, sem0, sem1):
    wid = _wid()
    neg = jnp.full((16,), NEG, jnp.float32)
    iota16 = lax.iota(jnp.int32, 16)
    sems = (sem0, sem1)

    def initr(r, carry):
        for k in range(8):
            acc[r, pl.ds(k * 16, 16)] = neg
        return carry

    lax.fori_loop(0, NB + 1, initr, 0)
    pltpu.sync_copy(cnt_hbm.at[pl.ds(pl.multiple_of(wid * 8, 8), 8)], cbuf.at[pl.ds(0, 8)])
    cnt = cbuf[pl.ds(0, 16)][0]
    nch = lax.div(cnt + (CHS - 1), CHS)

    def start(c, b):
        off = c * CHS
        pltpu.sync_copy(lst_hbm.at[pl.ds(pl.multiple_of(wid * CAP + off, 8), CHS)], lbuf.at[pl.ds(b * CHS, CHS)])
        for j in range(CHS // 16):
            lv = lbuf[pl.ds(b * CHS + j * 16, 16)]
            dl = lv & 511
            valid = (off + j * 16 + iota16) < cnt
            idbuf[pl.ds(b * CHS + j * 16, 16)] = lax.shift_right_logical(lv, 9)
            dlbuf[pl.ds(b * (CHS + 16) + j * 16, 16)] = jnp.where(valid, dl, NB)
        pltpu.async_copy(h_hbm.at[idbuf.at[pl.ds(b * CHS, CHS)]], rows.at[b], sems[b])

    def finish(b):
        pltpu.make_async_copy(h_hbm.at[idbuf.at[pl.ds(b * CHS, CHS)]], rows.at[b], sems[b]).wait()

        def edge(e4, carry2):
            e = e4 * 4
            lds = [dlbuf[pl.ds(b * (CHS + 16) + e + u, 16)][0] for u in range(4)]
            rs = [[rows[b, e + u, pl.ds(k * 16, 16)] for k in range(8)] for u in range(4)]
            for u in range(4):
                for k in range(8):
                    s = pl.ds(k * 16, 16)
                    acc[lds[u], s] = jnp.maximum(acc[lds[u], s], rs[u][k])
            return carry2

        lax.fori_loop(0, CHS // 4, edge, 0)

    @pl.when(0 < nch)
    def _():
        start(0, 0)

    @pl.when(1 < nch)
    def _():
        start(1, 1)

    def pair(p, carry):
        for b in range(2):
            c = 2 * p + b

            @pl.when(c < nch)
            def _():
                finish(b)

                @pl.when(c + 2 < nch)
                def _():
                    start(c + 2, b)

        return carry

    lax.fori_loop(0, lax.div(nch + 1, 2), pair, 0)

    def finr(r, carry):
        for k in range(8):
            s = pl.ds(k * 16, 16)
            v = acc[r, s]
            acc[r, s] = jnp.where(v == NEG, 0.0, v)
        return carry

    lax.fori_loop(0, NB, finr, 0)
    pltpu.sync_copy(acc.at[pl.ds(0, NB)], out_hbm.at[pl.ds(wid * NB, NB)])


def _sc_scatter_max(h, lst, cnts):
    f = functools.partial(
        pl.kernel,
        out_type=jax.ShapeDtypeStruct((NPAD, D), jnp.float32),
        mesh=_sc_mesh,
        compiler_params=pltpu.CompilerParams(needs_layout_passes=False),
        scratch_types=[
            pltpu.VMEM((NB + 1, D), jnp.float32),
            pltpu.VMEM((2 * CHS,), jnp.int32),
            pltpu.VMEM((2 * CHS,), jnp.int32),
            pltpu.VMEM((2 * (CHS + 16),), jnp.int32),
            pltpu.VMEM((2, CHS, D), jnp.float32),
            pltpu.VMEM((16,), jnp.int32),
            pltpu.SemaphoreType.DMA,
            pltpu.SemaphoreType.DMA,
        ],
    )
    return f(_smax_body)(h, lst, cnts)


def _gadd_body(a_hbm, b_hbm, src_hbm, dst_hbm, p_hbm, sbuf, dbuf, ra, rb, sem0, sem1):
    wid = _wid()
    nchunk = E // CHG
    sems = (sem0, sem1)
    HG = CHG // 2

    def gathers(b):
        yield a_hbm.at[dbuf.at[pl.ds(b * CHG, HG)]], ra.at[b, pl.ds(0, HG)]
        yield a_hbm.at[dbuf.at[pl.ds(b * CHG + HG, HG)]], ra.at[b, pl.ds(HG, HG)]
        yield b_hbm.at[sbuf.at[pl.ds(b * CHG, HG)]], rb.at[b, pl.ds(0, HG)]
        yield b_hbm.at[sbuf.at[pl.ds(b * CHG + HG, HG)]], rb.at[b, pl.ds(HG, HG)]

    def start(c, b):
        off = c * CHG
        pltpu.sync_copy(dst_hbm.at[pl.ds(pl.multiple_of(off, 8), CHG)], dbuf.at[pl.ds(b * CHG, CHG)])
        pltpu.sync_copy(src_hbm.at[pl.ds(pl.multiple_of(off, 8), CHG)], sbuf.at[pl.ds(b * CHG, CHG)])
        for s_ref, d_ref in gathers(b):
            pltpu.async_copy(s_ref, d_ref, sems[b])

    def finish(c, b):
        for s_ref, d_ref in gathers(b):
            pltpu.make_async_copy(s_ref, d_ref, sems[b]).wait()

        def addr(r2, carry2):
            r = r2 * 2
            for u in range(2):
                for k2 in range(8):
                    s = pl.ds(k2 * 16, 16)
                    ra[b, r + u, s] = ra[b, r + u, s] + rb[b, r + u, s]
            return carry2

        lax.fori_loop(0, CHG // 2, addr, 0)
        pltpu.sync_copy(ra.at[b], p_hbm.at[pl.ds(c * CHG, CHG)])

    c0 = wid
    c1 = wid + NT

    @pl.when(c0 < nchunk)
    def _():
        start(c0, 0)

    @pl.when(c1 < nchunk)
    def _():
        start(c1, 1)

    def it(k, carry):
        for b in range(2):
            c = wid + (2 * k + b) * NT

            @pl.when(c < nchunk)
            def _():
                finish(c, b)

                @pl.when(c + 2 * NT < nchunk)
                def _():
                    start(c + 2 * NT, b)

        return carry

    nit = (nchunk // NT + 2) // 2
    lax.fori_loop(0, nit, it, 0)


def _sc_gather_add(a, b, src, dst):
    f = functools.partial(
        pl.kernel,
        out_type=jax.ShapeDtypeStruct((E, D), jnp.float32),
        mesh=_sc_mesh,
        compiler_params=pltpu.CompilerParams(needs_layout_passes=False),
        scratch_types=[
            pltpu.VMEM((2 * CHG,), jnp.int32),
            pltpu.VMEM((2 * CHG,), jnp.int32),
            pltpu.VMEM((2, CHG, D), jnp.float32),
            pltpu.VMEM((2, CHG, D), jnp.float32),
            pltpu.SemaphoreType.DMA,
            pltpu.SemaphoreType.DMA,
        ],
    )
    return f(_gadd_body)(a, b, src, dst)


def kernel(x, edge_index, edge_attr, W1a, b1a, W2a, b2a, g1, be1, W1b, b1b, W2b, b2b, g2, be2):
    src = edge_index[0]
    dst = edge_index[1]
    wd1 = W1a[:D] - W1a[D:]
    ws1 = W1a[D:]
    wd2 = W1b[:D] - W1b[D:]
    ws2 = W1b[D:]

    lst, cnts = _sc_bucket(dst)
    a1, b1 = _tc_node_linear(x, wd1, ws1, b1a)
    p1 = _sc_gather_add(a1, b1, src, dst)
    h1 = _tc_mlp(p1, W2a, b2a)
    m1 = _sc_scatter_max(h1, lst, cnts)[:N]
    a2, b2 = _tc_bn_mish_linear(m1, g1, be1, wd2, ws2, b1b)
    p2 = _sc_gather_add(a2, b2, src, dst)
    h2 = _tc_mlp(p2, W2b, b2b)
    m2 = _sc_scatter_max(h2, lst, cnts)[:N]
    out = _tc_bn_mish(m2, g2, be2)
    return (out, edge_index, edge_attr)


# speedup vs baseline: 3.3866x; 1.0390x over previous
import functools

import jax
import jax.numpy as jnp
from jax import lax
from jax.experimental import pallas as pl
from jax.experimental.pallas import tpu as pltpu
from jax.experimental.pallas import tpu_sc as plsc

N = 10000
E = 320000
D = 128
NT = 32
NB = 320
NPAD = NT * NB
FLUSH = 2048
CAP = E + FLUSH
CHS = 128
CHG = 160
NEG = float("-inf")

_sc_mesh = plsc.VectorSubcoreMesh(core_axis_name="c", subcore_axis_name="s")


def _wid():
    return lax.axis_index("s") * 2 + lax.axis_index("c")


def _mish(v):
    u = jnp.exp(jnp.minimum(v, 20.0))
    t = u * (u + 2.0)
    return v * t / (t + 2.0)


def _node_linear_body(x_ref, wd_ref, ws_ref, b_ref, a_ref, bb_ref):
    xb = x_ref[...]
    a_ref[...] = jnp.dot(xb, wd_ref[...], preferred_element_type=jnp.float32, precision=lax.Precision.HIGHEST) + b_ref[...]
    bb_ref[...] = jnp.dot(xb, ws_ref[...], preferred_element_type=jnp.float32, precision=lax.Precision.HIGHEST)


def _tc_node_linear(x, wd, ws, b):
    n = x.shape[0]
    blk = 1000
    grid = n // blk
    return pl.pallas_call(
        _node_linear_body,
        grid=(grid,),
        in_specs=[
            pl.BlockSpec((blk, D), lambda i: (i, 0)),
            pl.BlockSpec((D, D), lambda i: (0, 0)),
            pl.BlockSpec((D, D), lambda i: (0, 0)),
            pl.BlockSpec((1, D), lambda i: (0, 0)),
        ],
        out_specs=[
            pl.BlockSpec((blk, D), lambda i: (i, 0)),
            pl.BlockSpec((blk, D), lambda i: (i, 0)),
        ],
        out_shape=[
            jax.ShapeDtypeStruct((n, D), jnp.float32),
            jax.ShapeDtypeStruct((n, D), jnp.float32),
        ],
    )(x, wd, ws, b.reshape(1, D))


def _mlp_body(p_ref, w2_ref, b2_ref, h_ref):
    m = _mish(p_ref[...])
    h_ref[...] = jnp.dot(m, w2_ref[...], preferred_element_type=jnp.float32, precision=lax.Precision.HIGHEST) + b2_ref[...]


def _tc_mlp(p, w2, b2):
    blk = 1280
    grid = E // blk
    return pl.pallas_call(
        _mlp_body,
        grid=(grid,),
        in_specs=[
            pl.BlockSpec((blk, D), lambda i: (i, 0)),
            pl.BlockSpec((D, D), lambda i: (0, 0)),
            pl.BlockSpec((1, D), lambda i: (0, 0)),
        ],
        out_specs=pl.BlockSpec((blk, D), lambda i: (i, 0)),
        out_shape=jax.ShapeDtypeStruct((E, D), jnp.float32),
    )(p, w2, b2.reshape(1, D))


def _bn_mish_linear_body(h_ref, g_ref, be_ref, wd_ref, ws_ref, b_ref, a_ref, bb_ref):
    h = h_ref[...]
    mean = jnp.mean(h, axis=0, keepdims=True)
    var = jnp.mean((h - mean) ** 2, axis=0, keepdims=True)
    hn = (h - mean) * lax.rsqrt(var + 1e-5) * g_ref[...] + be_ref[...]
    hm = _mish(hn)
    a_ref[...] = jnp.dot(hm, wd_ref[...], preferred_element_type=jnp.float32, precision=lax.Precision.HIGHEST) + b_ref[...]
    bb_ref[...] = jnp.dot(hm, ws_ref[...], preferred_element_type=jnp.float32, precision=lax.Precision.HIGHEST)


def _tc_bn_mish_linear(h, g, be, wd, ws, b):
    return pl.pallas_call(
        _bn_mish_linear_body,
        out_shape=[
            jax.ShapeDtypeStruct((N, D), jnp.float32),
            jax.ShapeDtypeStruct((N, D), jnp.float32),
        ],
    )(h, g.reshape(1, D), be.reshape(1, D), wd, ws, b.reshape(1, D))


def _bn_mish_body(h_ref, g_ref, be_ref, o_ref):
    h = h_ref[...]
    mean = jnp.mean(h, axis=0, keepdims=True)
    var = jnp.mean((h - mean) ** 2, axis=0, keepdims=True)
    hn = (h - mean) * lax.rsqrt(var + 1e-5) * g_ref[...] + be_ref[...]
    o_ref[...] = _mish(hn)


def _tc_bn_mish(h, g, be):
    return pl.pallas_call(
        _bn_mish_body,
        out_shape=jax.ShapeDtypeStruct((N, D), jnp.float32),
    )(h, g.reshape(1, D), be.reshape(1, D))


def _bucket_body(dst_hbm, lst_hbm, cnt_hbm, dstbuf, pbuf, cbuf):
    wid = _wid()
    lo = wid * NB
    zero = jnp.full((16,), NB, jnp.int32)

    def zstep(i, carry):
        pbuf[pl.ds(i * 16, 16)] = zero
        return carry

    lax.fori_loop(0, (FLUSH + 144) // 16, zstep, 0)
    iota16 = lax.iota(jnp.int32, 16)

    CH = 3200

    def chunk(c, carry):
        pltpu.sync_copy(dst_hbm.at[pl.ds(pl.multiple_of(c * CH, 8), CH)], dstbuf)

        def step(i, carry2):
            cur, written = carry2
            base = c * CH + i * 128
            vs = [dstbuf[pl.ds(i * 128 + u * 16, 16)] for u in range(8)]
            ms = [(v >= lo) & (v < lo + NB) for v in vs]
            cs = [jnp.cumsum(m.astype(jnp.int32)) for m in ms]
            cnts = [cc[15] for cc in cs]
            packs = [(iota16 + (base + u * 16)) * 512 + (vs[u] - lo) for u in range(8)]
            for u in range(8):
                plsc.store_scatter(pbuf, [cur + cs[u] - 1], packs[u], mask=ms[u])
                cur = cur + cnts[u]
            do = cur >= FLUSH

            @pl.when(do)
            def _():
                pltpu.sync_copy(pbuf.at[pl.ds(0, FLUSH)], lst_hbm.at[pl.ds(pl.multiple_of(wid * CAP + written, 8), FLUSH)])
                for u in range(9):
                    pbuf[pl.ds(u * 16, 16)] = pbuf[pl.ds(FLUSH + u * 16, 16)]

            cur = lax.select(do, cur - FLUSH, cur)
            written = lax.select(do, written + FLUSH, written)
            return (cur, written)

        return lax.fori_loop(0, CH // 128, step, carry)

    cur, written = lax.fori_loop(0, E // CH, chunk, (0, 0))
    pltpu.sync_copy(pbuf.at[pl.ds(0, FLUSH)], lst_hbm.at[pl.ds(pl.multiple_of(wid * CAP + written, 8), FLUSH)])
    cbuf[pl.ds(0, 16)] = jnp.zeros((16,), jnp.int32) + (written + cur)
    pltpu.sync_copy(cbuf.at[pl.ds(0, 8)], cnt_hbm.at[pl.ds(pl.multiple_of(wid * 8, 8), 8)])


def _sc_bucket(dst):
    f = functools.partial(
        pl.kernel,
        out_type=[
            jax.ShapeDtypeStruct((NT * CAP,), jnp.int32),
            jax.ShapeDtypeStruct((NT * 8,), jnp.int32),
        ],
        mesh=_sc_mesh,
        compiler_params=pltpu.CompilerParams(needs_layout_passes=False),
        scratch_types=[
            pltpu.VMEM((3200,), jnp.int32),
            pltpu.VMEM((FLUSH + 144,), jnp.int32),
            pltpu.VMEM((16,), jnp.int32),
        ],
    )
    return f(_bucket_body)(dst)


def _smax_body(h_hbm, lst_hbm, cnt_hbm, out_hbm, acc, lbuf, idbuf, dlbuf, rows, cbuf, sem0, sem1):
    wid = _wid()
    neg = jnp.full((16,), NEG, jnp.float32)
    iota16 = lax.iota(jnp.int32, 16)
    sems = (sem0, sem1)

    def initr(r, carry):
        for k in range(8):
            acc[r, pl.ds(k * 16, 16)] = neg
        return carry

    lax.fori_loop(0, NB + 1, initr, 0)
    pltpu.sync_copy(cnt_hbm.at[pl.ds(pl.multiple_of(wid * 8, 8), 8)], cbuf.at[pl.ds(0, 8)])
    cnt = cbuf[pl.ds(0, 16)][0]
    nch = lax.div(cnt + (CHS - 1), CHS)

    def start(c, b):
        off = c * CHS
        pltpu.sync_copy(lst_hbm.at[pl.ds(pl.multiple_of(wid * CAP + off, 8), CHS)], lbuf.at[pl.ds(b * CHS, CHS)])
        for j in range(CHS // 16):
            lv = lbuf[pl.ds(b * CHS + j * 16, 16)]
            dl = lv & 511
            valid = (off + j * 16 + iota16) < cnt
            idbuf[pl.ds(b * CHS + j * 16, 16)] = lax.shift_right_logical(lv, 9)
            dlbuf[pl.ds(b * (CHS + 16) + j * 16, 16)] = jnp.where(valid, dl, NB)
        pltpu.async_copy(h_hbm.at[idbuf.at[pl.ds(b * CHS, CHS)]], rows.at[b], sems[b])

    def finish(b):
        pltpu.make_async_copy(h_hbm.at[idbuf.at[pl.ds(b * CHS, CHS)]], rows.at[b], sems[b]).wait()

        def edge(e4, carry2):
            e = e4 * 4
            lds = [dlbuf[pl.ds(b * (CHS + 16) + e + u, 16)][0] for u in range(4)]
            rs = [[rows[b, e + u, pl.ds(k * 16, 16)] for k in range(8)] for u in range(4)]
            for u in range(4):
                for k in range(8):
                    s = pl.ds(k * 16, 16)
                    acc[lds[u], s] = jnp.maximum(acc[lds[u], s], rs[u][k])
            return carry2

        lax.fori_loop(0, CHS // 4, edge, 0)

    @pl.when(0 < nch)
    def _():
        start(0, 0)

    @pl.when(1 < nch)
    def _():
        start(1, 1)

    def pair(p, carry):
        for b in range(2):
            c = 2 * p + b

            @pl.when(c < nch)
            def _():
                finish(b)

                @pl.when(c + 2 < nch)
                def _():
                    start(c + 2, b)

        return carry

    lax.fori_loop(0, lax.div(nch + 1, 2), pair, 0)

    def finr(r, carry):
        for k in range(8):
            s = pl.ds(k * 16, 16)
            v = acc[r, s]
            acc[r, s] = jnp.where(v == NEG, 0.0, v)
        return carry

    lax.fori_loop(0, NB, finr, 0)
    pltpu.sync_copy(acc.at[pl.ds(0, NB)], out_hbm.at[pl.ds(wid * NB, NB)])


def _sc_scatter_max(h, lst, cnts):
    f = functools.partial(
        pl.kernel,
        out_type=jax.ShapeDtypeStruct((NPAD, D), jnp.float32),
        mesh=_sc_mesh,
        compiler_params=pltpu.CompilerParams(needs_layout_passes=False),
        scratch_types=[
            pltpu.VMEM((NB + 1, D), jnp.float32),
            pltpu.VMEM((2 * CHS,), jnp.int32),
            pltpu.VMEM((2 * CHS,), jnp.int32),
            pltpu.VMEM((2 * (CHS + 16),), jnp.int32),
            pltpu.VMEM((2, CHS, D), jnp.float32),
            pltpu.VMEM((16,), jnp.int32),
            pltpu.SemaphoreType.DMA,
            pltpu.SemaphoreType.DMA,
        ],
    )
    return f(_smax_body)(h, lst, cnts)


def _gadd_body(a_hbm, b_hbm, src_hbm, dst_hbm, p_hbm, sbuf, dbuf, ra, rb, sem0, sem1):
    wid = _wid()
    nchunk = E // CHG
    sems = (sem0, sem1)
    HG = CHG // 2

    def gathers(b):
        yield a_hbm.at[dbuf.at[pl.ds(b * CHG, HG)]], ra.at[b, pl.ds(0, HG)]
        yield a_hbm.at[dbuf.at[pl.ds(b * CHG + HG, HG)]], ra.at[b, pl.ds(HG, HG)]
        yield b_hbm.at[sbuf.at[pl.ds(b * CHG, HG)]], rb.at[b, pl.ds(0, HG)]
        yield b_hbm.at[sbuf.at[pl.ds(b * CHG + HG, HG)]], rb.at[b, pl.ds(HG, HG)]

    def start(c, b):
        off = c * CHG
        pltpu.sync_copy(dst_hbm.at[pl.ds(pl.multiple_of(off, 8), CHG)], dbuf.at[pl.ds(b * CHG, CHG)])
        pltpu.sync_copy(src_hbm.at[pl.ds(pl.multiple_of(off, 8), CHG)], sbuf.at[pl.ds(b * CHG, CHG)])
        for s_ref, d_ref in gathers(b):
            pltpu.async_copy(s_ref, d_ref, sems[b])

    def finish(c, b):
        for s_ref, d_ref in gathers(b):
            pltpu.make_async_copy(s_ref, d_ref, sems[b]).wait()

        def addr(r2, carry2):
            r = r2 * 2
            for u in range(2):
                for k2 in range(8):
                    s = pl.ds(k2 * 16, 16)
                    ra[b, r + u, s] = ra[b, r + u, s] + rb[b, r + u, s]
            return carry2

        lax.fori_loop(0, CHG // 2, addr, 0)
        pltpu.sync_copy(ra.at[b], p_hbm.at[pl.ds(c * CHG, CHG)])

    c0 = wid
    c1 = wid + NT

    @pl.when(c0 < nchunk)
    def _():
        start(c0, 0)

    @pl.when(c1 < nchunk)
    def _():
        start(c1, 1)

    def it(k, carry):
        for b in range(2):
            c = wid + (2 * k + b) * NT

            @pl.when(c < nchunk)
            def _():
                finish(c, b)

                @pl.when(c + 2 * NT < nchunk)
                def _():
                    start(c + 2 * NT, b)

        return carry

    nit = (nchunk // NT + 2) // 2
    lax.fori_loop(0, nit, it, 0)


def _sc_gather_add(a, b, src, dst):
    f = functools.partial(
        pl.kernel,
        out_type=jax.ShapeDtypeStruct((E, D), jnp.float32),
        mesh=_sc_mesh,
        compiler_params=pltpu.CompilerParams(needs_layout_passes=False),
        scratch_types=[
            pltpu.VMEM((2 * CHG,), jnp.int32),
            pltpu.VMEM((2 * CHG,), jnp.int32),
            pltpu.VMEM((2, CHG, D), jnp.float32),
            pltpu.VMEM((2, CHG, D), jnp.float32),
            pltpu.SemaphoreType.DMA,
            pltpu.SemaphoreType.DMA,
        ],
    )
    return f(_gadd_body)(a, b, src, dst)


def kernel(x, edge_index, edge_attr, W1a, b1a, W2a, b2a, g1, be1, W1b, b1b, W2b, b2b, g2, be2):
    src = edge_index[0]
    dst = edge_index[1]
    wd1 = W1a[:D] - W1a[D:]
    ws1 = W1a[D:]
    wd2 = W1b[:D] - W1b[D:]
    ws2 = W1b[D:]

    lst, cnts = _sc_bucket(dst)
    a1, b1 = _tc_node_linear(x, wd1, ws1, b1a)
    p1 = _sc_gather_add(a1, b1, src, dst)
    h1 = _tc_mlp(p1, W2a, b2a)
    m1 = _sc_scatter_max(h1, lst, cnts)[:N]
    a2, b2 = _tc_bn_mish_linear(m1, g1, be1, wd2, ws2, b1b)
    p2 = _sc_gather_add(a2, b2, src, dst)
    h2 = _tc_mlp(p2, W2b, b2b)
    m2 = _sc_scatter_max(h2, lst, cnts)[:N]
    out = _tc_bn_mish(m2, g2, be2)
    return (out, edge_index, edge_attr)


# speedup vs baseline: 3.6550x; 1.0793x over previous
import functools

import jax
import jax.numpy as jnp
from jax import lax
from jax.experimental import pallas as pl
from jax.experimental.pallas import tpu as pltpu
from jax.experimental.pallas import tpu_sc as plsc

N = 10000
E = 320000
D = 128
NT = 32
NB = 320
NPAD = NT * NB
FLUSH = 2048
CAP = E + FLUSH
CHS = 128
CHG = 160
NEG = float("-inf")

_sc_mesh = plsc.VectorSubcoreMesh(core_axis_name="c", subcore_axis_name="s")


def _wid():
    return lax.axis_index("s") * 2 + lax.axis_index("c")


def _mish(v):
    u = jnp.exp(jnp.minimum(v, 20.0))
    t = u * (u + 2.0)
    return v * t / (t + 2.0)


def _node_linear_body(x_ref, wd_ref, ws_ref, b_ref, a_ref, bb_ref):
    xb = x_ref[...]
    a_ref[...] = jnp.dot(xb, wd_ref[...], preferred_element_type=jnp.float32, precision=lax.Precision.HIGHEST) + b_ref[...]
    bb_ref[...] = jnp.dot(xb, ws_ref[...], preferred_element_type=jnp.float32, precision=lax.Precision.HIGHEST)


def _tc_node_linear(x, wd, ws, b):
    n = x.shape[0]
    blk = 1000
    grid = n // blk
    return pl.pallas_call(
        _node_linear_body,
        grid=(grid,),
        in_specs=[
            pl.BlockSpec((blk, D), lambda i: (i, 0)),
            pl.BlockSpec((D, D), lambda i: (0, 0)),
            pl.BlockSpec((D, D), lambda i: (0, 0)),
            pl.BlockSpec((1, D), lambda i: (0, 0)),
        ],
        out_specs=[
            pl.BlockSpec((blk, D), lambda i: (i, 0)),
            pl.BlockSpec((blk, D), lambda i: (i, 0)),
        ],
        out_shape=[
            jax.ShapeDtypeStruct((n, D), jnp.float32),
            jax.ShapeDtypeStruct((n, D), jnp.float32),
        ],
    )(x, wd, ws, b.reshape(1, D))


def _mlp_body(p_ref, w2_ref, b2_ref, h_ref):
    m = _mish(p_ref[...])
    h_ref[...] = jnp.dot(m, w2_ref[...], preferred_element_type=jnp.float32) + b2_ref[...]


def _tc_mlp(p, w2, b2):
    blk = 1280
    grid = E // blk
    return pl.pallas_call(
        _mlp_body,
        grid=(grid,),
        in_specs=[
            pl.BlockSpec((blk, D), lambda i: (i, 0)),
            pl.BlockSpec((D, D), lambda i: (0, 0)),
            pl.BlockSpec((1, D), lambda i: (0, 0)),
        ],
        out_specs=pl.BlockSpec((blk, D), lambda i: (i, 0)),
        out_shape=jax.ShapeDtypeStruct((E, D), jnp.float32),
    )(p, w2, b2.reshape(1, D))


def _bn_mish_linear_body(h_ref, g_ref, be_ref, wd_ref, ws_ref, b_ref, a_ref, bb_ref):
    h = h_ref[...]
    mean = jnp.mean(h, axis=0, keepdims=True)
    var = jnp.mean((h - mean) ** 2, axis=0, keepdims=True)
    hn = (h - mean) * lax.rsqrt(var + 1e-5) * g_ref[...] + be_ref[...]
    hm = _mish(hn)
    a_ref[...] = jnp.dot(hm, wd_ref[...], preferred_element_type=jnp.float32, precision=lax.Precision.HIGHEST) + b_ref[...]
    bb_ref[...] = jnp.dot(hm, ws_ref[...], preferred_element_type=jnp.float32, precision=lax.Precision.HIGHEST)


def _tc_bn_mish_linear(h, g, be, wd, ws, b):
    return pl.pallas_call(
        _bn_mish_linear_body,
        out_shape=[
            jax.ShapeDtypeStruct((N, D), jnp.float32),
            jax.ShapeDtypeStruct((N, D), jnp.float32),
        ],
    )(h, g.reshape(1, D), be.reshape(1, D), wd, ws, b.reshape(1, D))


def _bn_mish_body(h_ref, g_ref, be_ref, o_ref):
    h = h_ref[...]
    mean = jnp.mean(h, axis=0, keepdims=True)
    var = jnp.mean((h - mean) ** 2, axis=0, keepdims=True)
    hn = (h - mean) * lax.rsqrt(var + 1e-5) * g_ref[...] + be_ref[...]
    o_ref[...] = _mish(hn)


def _tc_bn_mish(h, g, be):
    return pl.pallas_call(
        _bn_mish_body,
        out_shape=jax.ShapeDtypeStruct((N, D), jnp.float32),
    )(h, g.reshape(1, D), be.reshape(1, D))


def _bucket_body(dst_hbm, lst_hbm, cnt_hbm, dstbuf, pbuf, cbuf):
    wid = _wid()
    lo = wid * NB
    zero = jnp.full((16,), NB, jnp.int32)

    def zstep(i, carry):
        pbuf[pl.ds(i * 16, 16)] = zero
        return carry

    lax.fori_loop(0, (FLUSH + 144) // 16, zstep, 0)
    iota16 = lax.iota(jnp.int32, 16)

    CH = 3200

    def chunk(c, carry):
        pltpu.sync_copy(dst_hbm.at[pl.ds(pl.multiple_of(c * CH, 8), CH)], dstbuf)

        def step(i, carry2):
            cur, written = carry2
            base = c * CH + i * 128
            vs = [dstbuf[pl.ds(i * 128 + u * 16, 16)] for u in range(8)]
            ms = [(v >= lo) & (v < lo + NB) for v in vs]
            cs = [jnp.cumsum(m.astype(jnp.int32)) for m in ms]
            cnts = [cc[15] for cc in cs]
            packs = [(iota16 + (base + u * 16)) * 512 + (vs[u] - lo) for u in range(8)]
            for u in range(8):
                plsc.store_scatter(pbuf, [cur + cs[u] - 1], packs[u], mask=ms[u])
                cur = cur + cnts[u]
            do = cur >= FLUSH

            @pl.when(do)
            def _():
                pltpu.sync_copy(pbuf.at[pl.ds(0, FLUSH)], lst_hbm.at[pl.ds(pl.multiple_of(wid * CAP + written, 8), FLUSH)])
                for u in range(9):
                    pbuf[pl.ds(u * 16, 16)] = pbuf[pl.ds(FLUSH + u * 16, 16)]

            cur = lax.select(do, cur - FLUSH, cur)
            written = lax.select(do, written + FLUSH, written)
            return (cur, written)

        return lax.fori_loop(0, CH // 128, step, carry)

    cur, written = lax.fori_loop(0, E // CH, chunk, (0, 0))
    pltpu.sync_copy(pbuf.at[pl.ds(0, FLUSH)], lst_hbm.at[pl.ds(pl.multiple_of(wid * CAP + written, 8), FLUSH)])
    cbuf[pl.ds(0, 16)] = jnp.zeros((16,), jnp.int32) + (written + cur)
    pltpu.sync_copy(cbuf.at[pl.ds(0, 8)], cnt_hbm.at[pl.ds(pl.multiple_of(wid * 8, 8), 8)])


def _sc_bucket(dst):
    f = functools.partial(
        pl.kernel,
        out_type=[
            jax.ShapeDtypeStruct((NT * CAP,), jnp.int32),
            jax.ShapeDtypeStruct((NT * 8,), jnp.int32),
        ],
        mesh=_sc_mesh,
        compiler_params=pltpu.CompilerParams(needs_layout_passes=False),
        scratch_types=[
            pltpu.VMEM((3200,), jnp.int32),
            pltpu.VMEM((FLUSH + 144,), jnp.int32),
            pltpu.VMEM((16,), jnp.int32),
        ],
    )
    return f(_bucket_body)(dst)


def _smax_body(h_hbm, lst_hbm, cnt_hbm, out_hbm, acc, lbuf, idbuf, dlbuf, rows, cbuf, sem0, sem1):
    wid = _wid()
    neg = jnp.full((16,), NEG, jnp.float32)
    iota16 = lax.iota(jnp.int32, 16)
    sems = (sem0, sem1)

    def initr(r, carry):
        for k in range(8):
            acc[r, pl.ds(k * 16, 16)] = neg
        return carry

    lax.fori_loop(0, NB + 1, initr, 0)
    pltpu.sync_copy(cnt_hbm.at[pl.ds(pl.multiple_of(wid * 8, 8), 8)], cbuf.at[pl.ds(0, 8)])
    cnt = cbuf[pl.ds(0, 16)][0]
    nch = lax.div(cnt + (CHS - 1), CHS)

    def start(c, b):
        off = c * CHS
        pltpu.sync_copy(lst_hbm.at[pl.ds(pl.multiple_of(wid * CAP + off, 8), CHS)], lbuf.at[pl.ds(b * CHS, CHS)])
        for j in range(CHS // 16):
            lv = lbuf[pl.ds(b * CHS + j * 16, 16)]
            dl = lv & 511
            valid = (off + j * 16 + iota16) < cnt
            idbuf[pl.ds(b * CHS + j * 16, 16)] = lax.shift_right_logical(lv, 9)
            dlbuf[pl.ds(b * (CHS + 16) + j * 16, 16)] = jnp.where(valid, dl, NB)
        pltpu.async_copy(h_hbm.at[idbuf.at[pl.ds(b * CHS, CHS)]], rows.at[b], sems[b])

    def finish(b):
        pltpu.make_async_copy(h_hbm.at[idbuf.at[pl.ds(b * CHS, CHS)]], rows.at[b], sems[b]).wait()

        def edge(e4, carry2):
            e = e4 * 4
            lds = [dlbuf[pl.ds(b * (CHS + 16) + e + u, 16)][0] for u in range(4)]
            rs = [[rows[b, e + u, pl.ds(k * 16, 16)] for k in range(8)] for u in range(4)]
            for u in range(4):
                for k in range(8):
                    s = pl.ds(k * 16, 16)
                    acc[lds[u], s] = jnp.maximum(acc[lds[u], s], rs[u][k])
            return carry2

        lax.fori_loop(0, CHS // 4, edge, 0)

    @pl.when(0 < nch)
    def _():
        start(0, 0)

    @pl.when(1 < nch)
    def _():
        start(1, 1)

    def pair(p, carry):
        for b in range(2):
            c = 2 * p + b

            @pl.when(c < nch)
            def _():
                finish(b)

                @pl.when(c + 2 < nch)
                def _():
                    start(c + 2, b)

        return carry

    lax.fori_loop(0, lax.div(nch + 1, 2), pair, 0)

    def finr(r, carry):
        for k in range(8):
            s = pl.ds(k * 16, 16)
            v = acc[r, s]
            acc[r, s] = jnp.where(v == NEG, 0.0, v)
        return carry

    lax.fori_loop(0, NB, finr, 0)
    pltpu.sync_copy(acc.at[pl.ds(0, NB)], out_hbm.at[pl.ds(wid * NB, NB)])


def _sc_scatter_max(h, lst, cnts):
    f = functools.partial(
        pl.kernel,
        out_type=jax.ShapeDtypeStruct((NPAD, D), jnp.float32),
        mesh=_sc_mesh,
        compiler_params=pltpu.CompilerParams(needs_layout_passes=False),
        scratch_types=[
            pltpu.VMEM((NB + 1, D), jnp.float32),
            pltpu.VMEM((2 * CHS,), jnp.int32),
            pltpu.VMEM((2 * CHS,), jnp.int32),
            pltpu.VMEM((2 * (CHS + 16),), jnp.int32),
            pltpu.VMEM((2, CHS, D), jnp.float32),
            pltpu.VMEM((16,), jnp.int32),
            pltpu.SemaphoreType.DMA,
            pltpu.SemaphoreType.DMA,
        ],
    )
    return f(_smax_body)(h, lst, cnts)


def _gadd_body(a_hbm, b_hbm, src_hbm, dst_hbm, p_hbm, sbuf, dbuf, ra, rb, sem0, sem1):
    wid = _wid()
    nchunk = E // CHG
    sems = (sem0, sem1)
    HG = CHG // 2

    def gathers(b):
        yield a_hbm.at[dbuf.at[pl.ds(b * CHG, HG)]], ra.at[b, pl.ds(0, HG)]
        yield a_hbm.at[dbuf.at[pl.ds(b * CHG + HG, HG)]], ra.at[b, pl.ds(HG, HG)]
        yield b_hbm.at[sbuf.at[pl.ds(b * CHG, HG)]], rb.at[b, pl.ds(0, HG)]
        yield b_hbm.at[sbuf.at[pl.ds(b * CHG + HG, HG)]], rb.at[b, pl.ds(HG, HG)]

    def start(c, b):
        off = c * CHG
        pltpu.sync_copy(dst_hbm.at[pl.ds(pl.multiple_of(off, 8), CHG)], dbuf.at[pl.ds(b * CHG, CHG)])
        pltpu.sync_copy(src_hbm.at[pl.ds(pl.multiple_of(off, 8), CHG)], sbuf.at[pl.ds(b * CHG, CHG)])
        for s_ref, d_ref in gathers(b):
            pltpu.async_copy(s_ref, d_ref, sems[b])

    def finish(c, b):
        for s_ref, d_ref in gathers(b):
            pltpu.make_async_copy(s_ref, d_ref, sems[b]).wait()

        def addr(r2, carry2):
            r = r2 * 2
            for u in range(2):
                for k2 in range(8):
                    s = pl.ds(k2 * 16, 16)
                    ra[b, r + u, s] = ra[b, r + u, s] + rb[b, r + u, s]
            return carry2

        lax.fori_loop(0, CHG // 2, addr, 0)
        pltpu.sync_copy(ra.at[b], p_hbm.at[pl.ds(c * CHG, CHG)])

    c0 = wid
    c1 = wid + NT

    @pl.when(c0 < nchunk)
    def _():
        start(c0, 0)

    @pl.when(c1 < nchunk)
    def _():
        start(c1, 1)

    def it(k, carry):
        for b in range(2):
            c = wid + (2 * k + b) * NT

            @pl.when(c < nchunk)
            def _():
                finish(c, b)

                @pl.when(c + 2 * NT < nchunk)
                def _():
                    start(c + 2 * NT, b)

        return carry

    nit = (nchunk // NT + 2) // 2
    lax.fori_loop(0, nit, it, 0)


def _sc_gather_add(a, b, src, dst):
    f = functools.partial(
        pl.kernel,
        out_type=jax.ShapeDtypeStruct((E, D), jnp.float32),
        mesh=_sc_mesh,
        compiler_params=pltpu.CompilerParams(needs_layout_passes=False),
        scratch_types=[
            pltpu.VMEM((2 * CHG,), jnp.int32),
            pltpu.VMEM((2 * CHG,), jnp.int32),
            pltpu.VMEM((2, CHG, D), jnp.float32),
            pltpu.VMEM((2, CHG, D), jnp.float32),
            pltpu.SemaphoreType.DMA,
            pltpu.SemaphoreType.DMA,
        ],
    )
    return f(_gadd_body)(a, b, src, dst)


def kernel(x, edge_index, edge_attr, W1a, b1a, W2a, b2a, g1, be1, W1b, b1b, W2b, b2b, g2, be2):
    src = edge_index[0]
    dst = edge_index[1]
    wd1 = W1a[:D] - W1a[D:]
    ws1 = W1a[D:]
    wd2 = W1b[:D] - W1b[D:]
    ws2 = W1b[D:]

    lst, cnts = _sc_bucket(dst)
    a1, b1 = _tc_node_linear(x, wd1, ws1, b1a)
    p1 = _sc_gather_add(a1, b1, src, dst)
    h1 = _tc_mlp(p1, W2a, b2a)
    m1 = _sc_scatter_max(h1, lst, cnts)[:N]
    a2, b2 = _tc_bn_mish_linear(m1, g1, be1, wd2, ws2, b1b)
    p2 = _sc_gather_add(a2, b2, src, dst)
    h2 = _tc_mlp(p2, W2b, b2b)
    m2 = _sc_scatter_max(h2, lst, cnts)[:N]
    out = _tc_bn_mish(m2, g2, be2)
    return (out, edge_index, edge_attr)


# speedup vs baseline: 3.8061x; 1.0413x over previous
import functools

import jax
import jax.numpy as jnp
from jax import lax
from jax.experimental import pallas as pl
from jax.experimental.pallas import tpu as pltpu
from jax.experimental.pallas import tpu_sc as plsc

N = 10000
E = 320000
D = 128
NT = 32
NB = 320
NPAD = NT * NB
FLUSH = 2048
CAP = E + FLUSH
CHS = 128
CHG = 160
NEG = float("-inf")

_sc_mesh = plsc.VectorSubcoreMesh(core_axis_name="c", subcore_axis_name="s")


def _wid():
    return lax.axis_index("s") * 2 + lax.axis_index("c")


def _mish(v):
    u = jnp.exp(jnp.minimum(v, 20.0))
    t = u * (u + 2.0)
    return v * t / (t + 2.0)


def _node_linear_body(x_ref, wd_ref, ws_ref, b_ref, a_ref, bb_ref):
    xb = x_ref[...]
    a_ref[...] = jnp.dot(xb, wd_ref[...], preferred_element_type=jnp.float32, precision=lax.Precision.HIGHEST) + b_ref[...]
    bb_ref[...] = jnp.dot(xb, ws_ref[...], preferred_element_type=jnp.float32, precision=lax.Precision.HIGHEST)


def _tc_node_linear(x, wd, ws, b):
    n = x.shape[0]
    blk = 1000
    grid = n // blk
    return pl.pallas_call(
        _node_linear_body,
        grid=(grid,),
        in_specs=[
            pl.BlockSpec((blk, D), lambda i: (i, 0)),
            pl.BlockSpec((D, D), lambda i: (0, 0)),
            pl.BlockSpec((D, D), lambda i: (0, 0)),
            pl.BlockSpec((1, D), lambda i: (0, 0)),
        ],
        out_specs=[
            pl.BlockSpec((blk, D), lambda i: (i, 0)),
            pl.BlockSpec((blk, D), lambda i: (i, 0)),
        ],
        out_shape=[
            jax.ShapeDtypeStruct((n, D), jnp.float32),
            jax.ShapeDtypeStruct((n, D), jnp.float32),
        ],
    )(x, wd, ws, b.reshape(1, D))


def _mlp_body(p_ref, w2_ref, b2_ref, h_ref):
    m = _mish(p_ref[...])
    h_ref[...] = jnp.dot(m, w2_ref[...], preferred_element_type=jnp.float32) + b2_ref[...]


def _tc_mlp(p, w2, b2):
    blk = 1280
    grid = E // blk
    return pl.pallas_call(
        _mlp_body,
        grid=(grid,),
        in_specs=[
            pl.BlockSpec((blk, D), lambda i: (i, 0)),
            pl.BlockSpec((D, D), lambda i: (0, 0)),
            pl.BlockSpec((1, D), lambda i: (0, 0)),
        ],
        out_specs=pl.BlockSpec((blk, D), lambda i: (i, 0)),
        out_shape=jax.ShapeDtypeStruct((E, D), jnp.float32),
    )(p, w2, b2.reshape(1, D))


def _tc_mlp_half(p, w2, b2, half, hprev=None):
    blk = 1280
    grid = (E // 2) // blk
    kwargs = {}
    ins = [p, w2, b2.reshape(1, D)]
    in_specs = [
        pl.BlockSpec((blk, D), lambda i: (i, 0)),
        pl.BlockSpec((D, D), lambda i: (0, 0)),
        pl.BlockSpec((1, D), lambda i: (0, 0)),
    ]
    if half == 0:
        body = _mlp_body
        out_index = lambda i: (i, 0)
    else:
        def body(h_ref, p_ref, w2_ref, b2_ref, o_ref):
            _mlp_body(p_ref, w2_ref, b2_ref, o_ref)
        ins = [hprev] + ins
        in_specs = [pl.BlockSpec(memory_space=pl.ANY)] + in_specs
        kwargs["input_output_aliases"] = {0: 0}
        out_index = lambda i: (grid + i, 0)
    return pl.pallas_call(
        body,
        grid=(grid,),
        in_specs=in_specs,
        out_specs=pl.BlockSpec((blk, D), out_index),
        out_shape=jax.ShapeDtypeStruct((E, D), jnp.float32),
        **kwargs,
    )(*ins)


def _bn_mish_linear_body(h_ref, g_ref, be_ref, wd_ref, ws_ref, b_ref, a_ref, bb_ref):
    h = h_ref[...]
    mean = jnp.mean(h, axis=0, keepdims=True)
    var = jnp.mean((h - mean) ** 2, axis=0, keepdims=True)
    hn = (h - mean) * lax.rsqrt(var + 1e-5) * g_ref[...] + be_ref[...]
    hm = _mish(hn)
    a_ref[...] = jnp.dot(hm, wd_ref[...], preferred_element_type=jnp.float32, precision=lax.Precision.HIGHEST) + b_ref[...]
    bb_ref[...] = jnp.dot(hm, ws_ref[...], preferred_element_type=jnp.float32, precision=lax.Precision.HIGHEST)


def _tc_bn_mish_linear(h, g, be, wd, ws, b):
    return pl.pallas_call(
        _bn_mish_linear_body,
        out_shape=[
            jax.ShapeDtypeStruct((N, D), jnp.float32),
            jax.ShapeDtypeStruct((N, D), jnp.float32),
        ],
    )(h, g.reshape(1, D), be.reshape(1, D), wd, ws, b.reshape(1, D))


def _bn_mish_body(h_ref, g_ref, be_ref, o_ref):
    h = h_ref[...]
    mean = jnp.mean(h, axis=0, keepdims=True)
    var = jnp.mean((h - mean) ** 2, axis=0, keepdims=True)
    hn = (h - mean) * lax.rsqrt(var + 1e-5) * g_ref[...] + be_ref[...]
    o_ref[...] = _mish(hn)


def _tc_bn_mish(h, g, be):
    return pl.pallas_call(
        _bn_mish_body,
        out_shape=jax.ShapeDtypeStruct((N, D), jnp.float32),
    )(h, g.reshape(1, D), be.reshape(1, D))


def _bucket_body(dst_hbm, lst_hbm, cnt_hbm, dstbuf, pbuf, cbuf):
    wid = _wid()
    lo = wid * NB
    zero = jnp.full((16,), NB, jnp.int32)

    def zstep(i, carry):
        pbuf[pl.ds(i * 16, 16)] = zero
        return carry

    lax.fori_loop(0, (FLUSH + 144) // 16, zstep, 0)
    iota16 = lax.iota(jnp.int32, 16)

    CH = 3200

    def chunk(c, carry):
        pltpu.sync_copy(dst_hbm.at[pl.ds(pl.multiple_of(c * CH, 8), CH)], dstbuf)

        def step(i, carry2):
            cur, written = carry2
            base = c * CH + i * 128
            vs = [dstbuf[pl.ds(i * 128 + u * 16, 16)] for u in range(8)]
            ms = [(v >= lo) & (v < lo + NB) for v in vs]
            cs = [jnp.cumsum(m.astype(jnp.int32)) for m in ms]
            cnts = [cc[15] for cc in cs]
            packs = [(iota16 + (base + u * 16)) * 512 + (vs[u] - lo) for u in range(8)]
            for u in range(8):
                plsc.store_scatter(pbuf, [cur + cs[u] - 1], packs[u], mask=ms[u])
                cur = cur + cnts[u]
            do = cur >= FLUSH

            @pl.when(do)
            def _():
                pltpu.sync_copy(pbuf.at[pl.ds(0, FLUSH)], lst_hbm.at[pl.ds(pl.multiple_of(wid * CAP + written, 8), FLUSH)])
                for u in range(9):
                    pbuf[pl.ds(u * 16, 16)] = pbuf[pl.ds(FLUSH + u * 16, 16)]

            cur = lax.select(do, cur - FLUSH, cur)
            written = lax.select(do, written + FLUSH, written)
            return (cur, written)

        return lax.fori_loop(0, CH // 128, step, carry)

    cur, written = lax.fori_loop(0, E // CH, chunk, (0, 0))
    pltpu.sync_copy(pbuf.at[pl.ds(0, FLUSH)], lst_hbm.at[pl.ds(pl.multiple_of(wid * CAP + written, 8), FLUSH)])
    cbuf[pl.ds(0, 16)] = jnp.zeros((16,), jnp.int32) + (written + cur)
    pltpu.sync_copy(cbuf.at[pl.ds(0, 8)], cnt_hbm.at[pl.ds(pl.multiple_of(wid * 8, 8), 8)])


def _sc_bucket(dst):
    f = functools.partial(
        pl.kernel,
        out_type=[
            jax.ShapeDtypeStruct((NT * CAP,), jnp.int32),
            jax.ShapeDtypeStruct((NT * 8,), jnp.int32),
        ],
        mesh=_sc_mesh,
        compiler_params=pltpu.CompilerParams(needs_layout_passes=False),
        scratch_types=[
            pltpu.VMEM((3200,), jnp.int32),
            pltpu.VMEM((FLUSH + 144,), jnp.int32),
            pltpu.VMEM((16,), jnp.int32),
        ],
    )
    return f(_bucket_body)(dst)


def _smax_body(h_hbm, lst_hbm, cnt_hbm, out_hbm, acc, lbuf, idbuf, dlbuf, rows, cbuf, sem0, sem1):
    wid = _wid()
    neg = jnp.full((16,), NEG, jnp.float32)
    iota16 = lax.iota(jnp.int32, 16)
    sems = (sem0, sem1)

    def initr(r, carry):
        for k in range(8):
            acc[r, pl.ds(k * 16, 16)] = neg
        return carry

    lax.fori_loop(0, NB + 1, initr, 0)
    pltpu.sync_copy(cnt_hbm.at[pl.ds(pl.multiple_of(wid * 8, 8), 8)], cbuf.at[pl.ds(0, 8)])
    cnt = cbuf[pl.ds(0, 16)][0]
    nch = lax.div(cnt + (CHS - 1), CHS)

    def start(c, b):
        off = c * CHS
        pltpu.sync_copy(lst_hbm.at[pl.ds(pl.multiple_of(wid * CAP + off, 8), CHS)], lbuf.at[pl.ds(b * CHS, CHS)])
        for j in range(CHS // 16):
            lv = lbuf[pl.ds(b * CHS + j * 16, 16)]
            dl = lv & 511
            valid = (off + j * 16 + iota16) < cnt
            idbuf[pl.ds(b * CHS + j * 16, 16)] = lax.shift_right_logical(lv, 9)
            dlbuf[pl.ds(b * (CHS + 16) + j * 16, 16)] = jnp.where(valid, dl, NB)
        pltpu.async_copy(h_hbm.at[idbuf.at[pl.ds(b * CHS, CHS)]], rows.at[b], sems[b])

    def finish(b):
        pltpu.make_async_copy(h_hbm.at[idbuf.at[pl.ds(b * CHS, CHS)]], rows.at[b], sems[b]).wait()

        def edge(e4, carry2):
            e = e4 * 4
            lds = [dlbuf[pl.ds(b * (CHS + 16) + e + u, 16)][0] for u in range(4)]
            rs = [[rows[b, e + u, pl.ds(k * 16, 16)] for k in range(8)] for u in range(4)]
            for u in range(4):
                for k in range(8):
                    s = pl.ds(k * 16, 16)
                    acc[lds[u], s] = jnp.maximum(acc[lds[u], s], rs[u][k])
            return carry2

        lax.fori_loop(0, CHS // 4, edge, 0)

    @pl.when(0 < nch)
    def _():
        start(0, 0)

    @pl.when(1 < nch)
    def _():
        start(1, 1)

    def pair(p, carry):
        for b in range(2):
            c = 2 * p + b

            @pl.when(c < nch)
            def _():
                finish(b)

                @pl.when(c + 2 < nch)
                def _():
                    start(c + 2, b)

        return carry

    lax.fori_loop(0, lax.div(nch + 1, 2), pair, 0)

    def finr(r, carry):
        for k in range(8):
            s = pl.ds(k * 16, 16)
            v = acc[r, s]
            acc[r, s] = jnp.where(v == NEG, 0.0, v)
        return carry

    lax.fori_loop(0, NB, finr, 0)
    pltpu.sync_copy(acc.at[pl.ds(0, NB)], out_hbm.at[pl.ds(wid * NB, NB)])


def _sc_scatter_max(h, lst, cnts):
    f = functools.partial(
        pl.kernel,
        out_type=jax.ShapeDtypeStruct((NPAD, D), jnp.float32),
        mesh=_sc_mesh,
        compiler_params=pltpu.CompilerParams(needs_layout_passes=False),
        scratch_types=[
            pltpu.VMEM((NB + 1, D), jnp.float32),
            pltpu.VMEM((2 * CHS,), jnp.int32),
            pltpu.VMEM((2 * CHS,), jnp.int32),
            pltpu.VMEM((2 * (CHS + 16),), jnp.int32),
            pltpu.VMEM((2, CHS, D), jnp.float32),
            pltpu.VMEM((16,), jnp.int32),
            pltpu.SemaphoreType.DMA,
            pltpu.SemaphoreType.DMA,
        ],
    )
    return f(_smax_body)(h, lst, cnts)


def _gadd_body(a_hbm, b_hbm, src_hbm, dst_hbm, p_hbm, sbuf, dbuf, ra, rb, sem0, sem1):
    wid = _wid()
    nchunk = p_hbm.shape[0] // CHG
    sems = (sem0, sem1)
    HG = CHG // 2

    def gathers(b):
        yield a_hbm.at[dbuf.at[pl.ds(b * CHG, HG)]], ra.at[b, pl.ds(0, HG)]
        yield a_hbm.at[dbuf.at[pl.ds(b * CHG + HG, HG)]], ra.at[b, pl.ds(HG, HG)]
        yield b_hbm.at[sbuf.at[pl.ds(b * CHG, HG)]], rb.at[b, pl.ds(0, HG)]
        yield b_hbm.at[sbuf.at[pl.ds(b * CHG + HG, HG)]], rb.at[b, pl.ds(HG, HG)]

    def start(c, b):
        off = c * CHG
        pltpu.sync_copy(dst_hbm.at[pl.ds(pl.multiple_of(off, 8), CHG)], dbuf.at[pl.ds(b * CHG, CHG)])
        pltpu.sync_copy(src_hbm.at[pl.ds(pl.multiple_of(off, 8), CHG)], sbuf.at[pl.ds(b * CHG, CHG)])
        for s_ref, d_ref in gathers(b):
            pltpu.async_copy(s_ref, d_ref, sems[b])

    def finish(c, b):
        for s_ref, d_ref in gathers(b):
            pltpu.make_async_copy(s_ref, d_ref, sems[b]).wait()

        def addr(r2, carry2):
            r = r2 * 2
            for u in range(2):
                for k2 in range(8):
                    s = pl.ds(k2 * 16, 16)
                    ra[b, r + u, s] = ra[b, r + u, s] + rb[b, r + u, s]
            return carry2

        lax.fori_loop(0, CHG // 2, addr, 0)
        pltpu.sync_copy(ra.at[b], p_hbm.at[pl.ds(c * CHG, CHG)])

    c0 = wid
    c1 = wid + NT

    @pl.when(c0 < nchunk)
    def _():
        start(c0, 0)

    @pl.when(c1 < nchunk)
    def _():
        start(c1, 1)

    def it(k, carry):
        for b in range(2):
            c = wid + (2 * k + b) * NT

            @pl.when(c < nchunk)
            def _():
                finish(c, b)

                @pl.when(c + 2 * NT < nchunk)
                def _():
                    start(c + 2 * NT, b)

        return carry

    nit = (nchunk // NT + 2) // 2
    lax.fori_loop(0, nit, it, 0)


def _sc_gather_add(a, b, src, dst):
    ne = src.shape[0]
    f = functools.partial(
        pl.kernel,
        out_type=jax.ShapeDtypeStruct((ne, D), jnp.float32),
        mesh=_sc_mesh,
        compiler_params=pltpu.CompilerParams(needs_layout_passes=False),
        scratch_types=[
            pltpu.VMEM((2 * CHG,), jnp.int32),
            pltpu.VMEM((2 * CHG,), jnp.int32),
            pltpu.VMEM((2, CHG, D), jnp.float32),
            pltpu.VMEM((2, CHG, D), jnp.float32),
            pltpu.SemaphoreType.DMA,
            pltpu.SemaphoreType.DMA,
        ],
    )
    return f(_gadd_body)(a, b, src, dst)


def kernel(x, edge_index, edge_attr, W1a, b1a, W2a, b2a, g1, be1, W1b, b1b, W2b, b2b, g2, be2):
    src = edge_index[0]
    dst = edge_index[1]
    wd1 = W1a[:D] - W1a[D:]
    ws1 = W1a[D:]
    wd2 = W1b[:D] - W1b[D:]
    ws2 = W1b[D:]

    lst, cnts = _sc_bucket(dst)
    a1, b1 = _tc_node_linear(x, wd1, ws1, b1a)
    p1 = _sc_gather_add(a1, b1, src, dst)
    h1 = _tc_mlp(p1, W2a, b2a)
    m1 = _sc_scatter_max(h1, lst, cnts)[:N]
    a2, b2 = _tc_bn_mish_linear(m1, g1, be1, wd2, ws2, b1b)
    e2 = E // 2
    p2a = _sc_gather_add(a2, b2, src[:e2], dst[:e2])
    p2b = _sc_gather_add(a2, b2, src[e2:], dst[e2:])
    h2a = _tc_mlp_half(p2a, W2b, b2b, 0)
    h2 = _tc_mlp_half(p2b, W2b, b2b, 1, h2a)
    m2 = _sc_scatter_max(h2, lst, cnts)[:N]
    out = _tc_bn_mish(m2, g2, be2)
    return (out, edge_index, edge_attr)


# speedup vs baseline: 3.8881x; 1.0215x over previous
import functools

import jax
import jax.numpy as jnp
from jax import lax
from jax.experimental import pallas as pl
from jax.experimental.pallas import tpu as pltpu
from jax.experimental.pallas import tpu_sc as plsc

N = 10000
E = 320000
D = 128
NT = 32
NB = 320
NPAD = NT * NB
FLUSH = 2048
CAP = E + FLUSH
CHS = 128
CHG = 160
NEG = float("-inf")

_sc_mesh = plsc.VectorSubcoreMesh(core_axis_name="c", subcore_axis_name="s")


def _wid():
    return lax.axis_index("s") * 2 + lax.axis_index("c")


def _mish(v):
    u = jnp.exp(jnp.minimum(v, 20.0))
    t = u * (u + 2.0)
    return v * t / (t + 2.0)


def _node_linear_body(x_ref, wd_ref, ws_ref, b_ref, a_ref, bb_ref):
    xb = x_ref[...]
    a_ref[...] = jnp.dot(xb, wd_ref[...], preferred_element_type=jnp.float32, precision=lax.Precision.HIGHEST) + b_ref[...]
    bb_ref[...] = jnp.dot(xb, ws_ref[...], preferred_element_type=jnp.float32, precision=lax.Precision.HIGHEST)


def _tc_node_linear(x, wd, ws, b):
    n = x.shape[0]
    blk = 1000
    grid = n // blk
    return pl.pallas_call(
        _node_linear_body,
        grid=(grid,),
        in_specs=[
            pl.BlockSpec((blk, D), lambda i: (i, 0)),
            pl.BlockSpec((D, D), lambda i: (0, 0)),
            pl.BlockSpec((D, D), lambda i: (0, 0)),
            pl.BlockSpec((1, D), lambda i: (0, 0)),
        ],
        out_specs=[
            pl.BlockSpec((blk, D), lambda i: (i, 0)),
            pl.BlockSpec((blk, D), lambda i: (i, 0)),
        ],
        out_shape=[
            jax.ShapeDtypeStruct((n, D), jnp.float32),
            jax.ShapeDtypeStruct((n, D), jnp.float32),
        ],
    )(x, wd, ws, b.reshape(1, D))


def _mlp_body(p_ref, w2_ref, b2_ref, h_ref):
    m = _mish(p_ref[...])
    h_ref[...] = jnp.dot(m, w2_ref[...], preferred_element_type=jnp.float32) + b2_ref[...]


def _tc_mlp(p, w2, b2):
    blk = 1280
    grid = E // blk
    return pl.pallas_call(
        _mlp_body,
        grid=(grid,),
        in_specs=[
            pl.BlockSpec((blk, D), lambda i: (i, 0)),
            pl.BlockSpec((D, D), lambda i: (0, 0)),
            pl.BlockSpec((1, D), lambda i: (0, 0)),
        ],
        out_specs=pl.BlockSpec((blk, D), lambda i: (i, 0)),
        out_shape=jax.ShapeDtypeStruct((E, D), jnp.float32),
    )(p, w2, b2.reshape(1, D))


def _tc_mlp_part(p, w2, b2, part, nparts, hprev=None):
    blk = 1280
    grid = (E // nparts) // blk
    base = part * grid
    kwargs = {}
    ins = [p, w2, b2.reshape(1, D)]
    in_specs = [
        pl.BlockSpec((blk, D), lambda i: (i, 0)),
        pl.BlockSpec((D, D), lambda i: (0, 0)),
        pl.BlockSpec((1, D), lambda i: (0, 0)),
    ]
    if part == 0:
        body = _mlp_body
    else:
        def body(h_ref, p_ref, w2_ref, b2_ref, o_ref):
            _mlp_body(p_ref, w2_ref, b2_ref, o_ref)
        ins = [hprev] + ins
        in_specs = [pl.BlockSpec(memory_space=pl.ANY)] + in_specs
        kwargs["input_output_aliases"] = {0: 0}
    return pl.pallas_call(
        body,
        grid=(grid,),
        in_specs=in_specs,
        out_specs=pl.BlockSpec((blk, D), lambda i: (base + i, 0)),
        out_shape=jax.ShapeDtypeStruct((E, D), jnp.float32),
        **kwargs,
    )(*ins)


def _bn_mish_linear_body(h_ref, g_ref, be_ref, wd_ref, ws_ref, b_ref, a_ref, bb_ref):
    h = h_ref[...]
    mean = jnp.mean(h, axis=0, keepdims=True)
    var = jnp.mean((h - mean) ** 2, axis=0, keepdims=True)
    hn = (h - mean) * lax.rsqrt(var + 1e-5) * g_ref[...] + be_ref[...]
    hm = _mish(hn)
    a_ref[...] = jnp.dot(hm, wd_ref[...], preferred_element_type=jnp.float32, precision=lax.Precision.HIGHEST) + b_ref[...]
    bb_ref[...] = jnp.dot(hm, ws_ref[...], preferred_element_type=jnp.float32, precision=lax.Precision.HIGHEST)


def _tc_bn_mish_linear(h, g, be, wd, ws, b):
    return pl.pallas_call(
        _bn_mish_linear_body,
        out_shape=[
            jax.ShapeDtypeStruct((N, D), jnp.float32),
            jax.ShapeDtypeStruct((N, D), jnp.float32),
        ],
    )(h, g.reshape(1, D), be.reshape(1, D), wd, ws, b.reshape(1, D))


def _bn_mish_body(h_ref, g_ref, be_ref, o_ref):
    h = h_ref[...]
    mean = jnp.mean(h, axis=0, keepdims=True)
    var = jnp.mean((h - mean) ** 2, axis=0, keepdims=True)
    hn = (h - mean) * lax.rsqrt(var + 1e-5) * g_ref[...] + be_ref[...]
    o_ref[...] = _mish(hn)


def _tc_bn_mish(h, g, be):
    return pl.pallas_call(
        _bn_mish_body,
        out_shape=jax.ShapeDtypeStruct((N, D), jnp.float32),
    )(h, g.reshape(1, D), be.reshape(1, D))


def _bucket_body(dst_hbm, lst_hbm, cnt_hbm, dstbuf, pbuf, cbuf):
    wid = _wid()
    lo = wid * NB
    zero = jnp.full((16,), NB, jnp.int32)

    def zstep(i, carry):
        pbuf[pl.ds(i * 16, 16)] = zero
        return carry

    lax.fori_loop(0, (FLUSH + 144) // 16, zstep, 0)
    iota16 = lax.iota(jnp.int32, 16)

    CH = 3200

    def chunk(c, carry):
        pltpu.sync_copy(dst_hbm.at[pl.ds(pl.multiple_of(c * CH, 8), CH)], dstbuf)

        def step(i, carry2):
            cur, written = carry2
            base = c * CH + i * 128
            vs = [dstbuf[pl.ds(i * 128 + u * 16, 16)] for u in range(8)]
            ms = [(v >= lo) & (v < lo + NB) for v in vs]
            cs = [jnp.cumsum(m.astype(jnp.int32)) for m in ms]
            cnts = [cc[15] for cc in cs]
            packs = [(iota16 + (base + u * 16)) * 512 + (vs[u] - lo) for u in range(8)]
            for u in range(8):
                plsc.store_scatter(pbuf, [cur + cs[u] - 1], packs[u], mask=ms[u])
                cur = cur + cnts[u]
            do = cur >= FLUSH

            @pl.when(do)
            def _():
                pltpu.sync_copy(pbuf.at[pl.ds(0, FLUSH)], lst_hbm.at[pl.ds(pl.multiple_of(wid * CAP + written, 8), FLUSH)])
                for u in range(9):
                    pbuf[pl.ds(u * 16, 16)] = pbuf[pl.ds(FLUSH + u * 16, 16)]

            cur = lax.select(do, cur - FLUSH, cur)
            written = lax.select(do, written + FLUSH, written)
            return (cur, written)

        return lax.fori_loop(0, CH // 128, step, carry)

    cur, written = lax.fori_loop(0, E // CH, chunk, (0, 0))
    pltpu.sync_copy(pbuf.at[pl.ds(0, FLUSH)], lst_hbm.at[pl.ds(pl.multiple_of(wid * CAP + written, 8), FLUSH)])
    cbuf[pl.ds(0, 16)] = jnp.zeros((16,), jnp.int32) + (written + cur)
    pltpu.sync_copy(cbuf.at[pl.ds(0, 8)], cnt_hbm.at[pl.ds(pl.multiple_of(wid * 8, 8), 8)])


def _sc_bucket(dst):
    f = functools.partial(
        pl.kernel,
        out_type=[
            jax.ShapeDtypeStruct((NT * CAP,), jnp.int32),
            jax.ShapeDtypeStruct((NT * 8,), jnp.int32),
        ],
        mesh=_sc_mesh,
        compiler_params=pltpu.CompilerParams(needs_layout_passes=False),
        scratch_types=[
            pltpu.VMEM((3200,), jnp.int32),
            pltpu.VMEM((FLUSH + 144,), jnp.int32),
            pltpu.VMEM((16,), jnp.int32),
        ],
    )
    return f(_bucket_body)(dst)


def _smax_body(h_hbm, lst_hbm, cnt_hbm, out_hbm, acc, lbuf, idbuf, dlbuf, rows, cbuf, sem0, sem1):
    wid = _wid()
    neg = jnp.full((16,), NEG, jnp.float32)
    iota16 = lax.iota(jnp.int32, 16)
    sems = (sem0, sem1)

    def initr(r, carry):
        for k in range(8):
            acc[r, pl.ds(k * 16, 16)] = neg
        return carry

    lax.fori_loop(0, NB + 1, initr, 0)
    pltpu.sync_copy(cnt_hbm.at[pl.ds(pl.multiple_of(wid * 8, 8), 8)], cbuf.at[pl.ds(0, 8)])
    cnt = cbuf[pl.ds(0, 16)][0]
    nch = lax.div(cnt + (CHS - 1), CHS)

    def start(c, b):
        off = c * CHS
        pltpu.sync_copy(lst_hbm.at[pl.ds(pl.multiple_of(wid * CAP + off, 8), CHS)], lbuf.at[pl.ds(b * CHS, CHS)])
        for j in range(CHS // 16):
            lv = lbuf[pl.ds(b * CHS + j * 16, 16)]
            dl = lv & 511
            valid = (off + j * 16 + iota16) < cnt
            idbuf[pl.ds(b * CHS + j * 16, 16)] = lax.shift_right_logical(lv, 9)
            dlbuf[pl.ds(b * (CHS + 16) + j * 16, 16)] = jnp.where(valid, dl, NB)
        pltpu.async_copy(h_hbm.at[idbuf.at[pl.ds(b * CHS, CHS)]], rows.at[b], sems[b])

    def finish(b):
        pltpu.make_async_copy(h_hbm.at[idbuf.at[pl.ds(b * CHS, CHS)]], rows.at[b], sems[b]).wait()

        def edge(e4, carry2):
            e = e4 * 4
            lds = [dlbuf[pl.ds(b * (CHS + 16) + e + u, 16)][0] for u in range(4)]
            rs = [[rows[b, e + u, pl.ds(k * 16, 16)] for k in range(8)] for u in range(4)]
            for u in range(4):
                for k in range(8):
                    s = pl.ds(k * 16, 16)
                    acc[lds[u], s] = jnp.maximum(acc[lds[u], s], rs[u][k])
            return carry2

        lax.fori_loop(0, CHS // 4, edge, 0)

    @pl.when(0 < nch)
    def _():
        start(0, 0)

    @pl.when(1 < nch)
    def _():
        start(1, 1)

    def pair(p, carry):
        for b in range(2):
            c = 2 * p + b

            @pl.when(c < nch)
            def _():
                finish(b)

                @pl.when(c + 2 < nch)
                def _():
                    start(c + 2, b)

        return carry

    lax.fori_loop(0, lax.div(nch + 1, 2), pair, 0)

    def finr(r, carry):
        for k in range(8):
            s = pl.ds(k * 16, 16)
            v = acc[r, s]
            acc[r, s] = jnp.where(v == NEG, 0.0, v)
        return carry

    lax.fori_loop(0, NB, finr, 0)
    pltpu.sync_copy(acc.at[pl.ds(0, NB)], out_hbm.at[pl.ds(wid * NB, NB)])


def _sc_scatter_max(h, lst, cnts):
    f = functools.partial(
        pl.kernel,
        out_type=jax.ShapeDtypeStruct((NPAD, D), jnp.float32),
        mesh=_sc_mesh,
        compiler_params=pltpu.CompilerParams(needs_layout_passes=False),
        scratch_types=[
            pltpu.VMEM((NB + 1, D), jnp.float32),
            pltpu.VMEM((2 * CHS,), jnp.int32),
            pltpu.VMEM((2 * CHS,), jnp.int32),
            pltpu.VMEM((2 * (CHS + 16),), jnp.int32),
            pltpu.VMEM((2, CHS, D), jnp.float32),
            pltpu.VMEM((16,), jnp.int32),
            pltpu.SemaphoreType.DMA,
            pltpu.SemaphoreType.DMA,
        ],
    )
    return f(_smax_body)(h, lst, cnts)


def _gadd_body(a_hbm, b_hbm, src_hbm, dst_hbm, p_hbm, sbuf, dbuf, ra, rb, sem0, sem1):
    wid = _wid()
    nchunk = p_hbm.shape[0] // CHG
    sems = (sem0, sem1)
    HG = CHG // 2

    def gathers(b):
        yield a_hbm.at[dbuf.at[pl.ds(b * CHG, HG)]], ra.at[b, pl.ds(0, HG)]
        yield a_hbm.at[dbuf.at[pl.ds(b * CHG + HG, HG)]], ra.at[b, pl.ds(HG, HG)]
        yield b_hbm.at[sbuf.at[pl.ds(b * CHG, HG)]], rb.at[b, pl.ds(0, HG)]
        yield b_hbm.at[sbuf.at[pl.ds(b * CHG + HG, HG)]], rb.at[b, pl.ds(HG, HG)]

    def start(c, b):
        off = c * CHG
        pltpu.sync_copy(dst_hbm.at[pl.ds(pl.multiple_of(off, 8), CHG)], dbuf.at[pl.ds(b * CHG, CHG)])
        pltpu.sync_copy(src_hbm.at[pl.ds(pl.multiple_of(off, 8), CHG)], sbuf.at[pl.ds(b * CHG, CHG)])
        for s_ref, d_ref in gathers(b):
            pltpu.async_copy(s_ref, d_ref, sems[b])

    def finish(c, b):
        for s_ref, d_ref in gathers(b):
            pltpu.make_async_copy(s_ref, d_ref, sems[b]).wait()

        def addr(r2, carry2):
            r = r2 * 2
            for u in range(2):
                for k2 in range(8):
                    s = pl.ds(k2 * 16, 16)
                    ra[b, r + u, s] = ra[b, r + u, s] + rb[b, r + u, s]
            return carry2

        lax.fori_loop(0, CHG // 2, addr, 0)
        pltpu.sync_copy(ra.at[b], p_hbm.at[pl.ds(c * CHG, CHG)])

    c0 = wid
    c1 = wid + NT

    @pl.when(c0 < nchunk)
    def _():
        start(c0, 0)

    @pl.when(c1 < nchunk)
    def _():
        start(c1, 1)

    def it(k, carry):
        for b in range(2):
            c = wid + (2 * k + b) * NT

            @pl.when(c < nchunk)
            def _():
                finish(c, b)

                @pl.when(c + 2 * NT < nchunk)
                def _():
                    start(c + 2 * NT, b)

        return carry

    nit = (nchunk // NT + 2) // 2
    lax.fori_loop(0, nit, it, 0)


def _sc_gather_add(a, b, src, dst):
    ne = src.shape[0]
    f = functools.partial(
        pl.kernel,
        out_type=jax.ShapeDtypeStruct((ne, D), jnp.float32),
        mesh=_sc_mesh,
        compiler_params=pltpu.CompilerParams(needs_layout_passes=False),
        scratch_types=[
            pltpu.VMEM((2 * CHG,), jnp.int32),
            pltpu.VMEM((2 * CHG,), jnp.int32),
            pltpu.VMEM((2, CHG, D), jnp.float32),
            pltpu.VMEM((2, CHG, D), jnp.float32),
            pltpu.SemaphoreType.DMA,
            pltpu.SemaphoreType.DMA,
        ],
    )
    return f(_gadd_body)(a, b, src, dst)


def kernel(x, edge_index, edge_attr, W1a, b1a, W2a, b2a, g1, be1, W1b, b1b, W2b, b2b, g2, be2):
    src = edge_index[0]
    dst = edge_index[1]
    wd1 = W1a[:D] - W1a[D:]
    ws1 = W1a[D:]
    wd2 = W1b[:D] - W1b[D:]
    ws2 = W1b[D:]

    lst, cnts = _sc_bucket(dst)
    a1, b1 = _tc_node_linear(x, wd1, ws1, b1a)
    p1 = _sc_gather_add(a1, b1, src, dst)
    h1 = _tc_mlp(p1, W2a, b2a)
    m1 = _sc_scatter_max(h1, lst, cnts)[:N]
    a2, b2 = _tc_bn_mish_linear(m1, g1, be1, wd2, ws2, b1b)
    nparts = 4
    ep = E // nparts
    h2 = None
    for q in range(nparts):
        pq = _sc_gather_add(a2, b2, src[q * ep:(q + 1) * ep], dst[q * ep:(q + 1) * ep])
        h2 = _tc_mlp_part(pq, W2b, b2b, q, nparts, h2)
    m2 = _sc_scatter_max(h2, lst, cnts)[:N]
    out = _tc_bn_mish(m2, g2, be2)
    return (out, edge_index, edge_attr)


# speedup vs baseline: 3.9007x; 1.0032x over previous
import functools

import jax
import jax.numpy as jnp
from jax import lax
from jax.experimental import pallas as pl
from jax.experimental.pallas import tpu as pltpu
from jax.experimental.pallas import tpu_sc as plsc

N = 10000
E = 320000
D = 128
NT = 32
NB = 320
NPAD = NT * NB
FLUSH = 2048
CAP = E + FLUSH
CHS = 128
CHG = 160
NEG = float("-inf")

_sc_mesh = plsc.VectorSubcoreMesh(core_axis_name="c", subcore_axis_name="s")


def _wid():
    return lax.axis_index("s") * 2 + lax.axis_index("c")


def _mish(v):
    u = jnp.exp(jnp.minimum(v, 20.0))
    t = u * (u + 2.0)
    return v * t / (t + 2.0)


def _node_linear_body(x_ref, wd_ref, ws_ref, b_ref, a_ref, bb_ref):
    xb = x_ref[...]
    a_ref[...] = jnp.dot(xb, wd_ref[...], preferred_element_type=jnp.float32, precision=lax.Precision.HIGHEST) + b_ref[...]
    bb_ref[...] = jnp.dot(xb, ws_ref[...], preferred_element_type=jnp.float32, precision=lax.Precision.HIGHEST)


def _tc_node_linear(x, wd, ws, b):
    n = x.shape[0]
    blk = 1000
    grid = n // blk
    return pl.pallas_call(
        _node_linear_body,
        grid=(grid,),
        in_specs=[
            pl.BlockSpec((blk, D), lambda i: (i, 0)),
            pl.BlockSpec((D, D), lambda i: (0, 0)),
            pl.BlockSpec((D, D), lambda i: (0, 0)),
            pl.BlockSpec((1, D), lambda i: (0, 0)),
        ],
        out_specs=[
            pl.BlockSpec((blk, D), lambda i: (i, 0)),
            pl.BlockSpec((blk, D), lambda i: (i, 0)),
        ],
        out_shape=[
            jax.ShapeDtypeStruct((n, D), jnp.float32),
            jax.ShapeDtypeStruct((n, D), jnp.float32),
        ],
    )(x, wd, ws, b.reshape(1, D))


def _mlp_body(p_ref, w2_ref, b2_ref, h_ref):
    m = _mish(p_ref[...])
    h_ref[...] = jnp.dot(m, w2_ref[...], preferred_element_type=jnp.float32) + b2_ref[...]


def _tc_mlp(p, w2, b2):
    blk = 1280
    grid = E // blk
    return pl.pallas_call(
        _mlp_body,
        grid=(grid,),
        in_specs=[
            pl.BlockSpec((blk, D), lambda i: (i, 0)),
            pl.BlockSpec((D, D), lambda i: (0, 0)),
            pl.BlockSpec((1, D), lambda i: (0, 0)),
        ],
        out_specs=pl.BlockSpec((blk, D), lambda i: (i, 0)),
        out_shape=jax.ShapeDtypeStruct((E, D), jnp.float32),
    )(p, w2, b2.reshape(1, D))


def _tc_mlp_part(p, w2, b2, part, nparts, hprev=None):
    blk = 1280
    grid = (E // nparts) // blk
    base = part * grid
    kwargs = {}
    ins = [p, w2, b2.reshape(1, D)]
    in_specs = [
        pl.BlockSpec((blk, D), lambda i: (i, 0)),
        pl.BlockSpec((D, D), lambda i: (0, 0)),
        pl.BlockSpec((1, D), lambda i: (0, 0)),
    ]
    if part == 0:
        body = _mlp_body
    else:
        def body(h_ref, p_ref, w2_ref, b2_ref, o_ref):
            _mlp_body(p_ref, w2_ref, b2_ref, o_ref)
        ins = [hprev] + ins
        in_specs = [pl.BlockSpec(memory_space=pl.ANY)] + in_specs
        kwargs["input_output_aliases"] = {0: 0}
    return pl.pallas_call(
        body,
        grid=(grid,),
        in_specs=in_specs,
        out_specs=pl.BlockSpec((blk, D), lambda i: (base + i, 0)),
        out_shape=jax.ShapeDtypeStruct((E, D), jnp.float32),
        **kwargs,
    )(*ins)


def _bn_mish_linear_body(h_ref, g_ref, be_ref, wd_ref, ws_ref, b_ref, a_ref, bb_ref):
    h = h_ref[...]
    mean = jnp.mean(h, axis=0, keepdims=True)
    var = jnp.mean((h - mean) ** 2, axis=0, keepdims=True)
    hn = (h - mean) * lax.rsqrt(var + 1e-5) * g_ref[...] + be_ref[...]
    hm = _mish(hn)
    a_ref[...] = jnp.dot(hm, wd_ref[...], preferred_element_type=jnp.float32, precision=lax.Precision.HIGHEST) + b_ref[...]
    bb_ref[...] = jnp.dot(hm, ws_ref[...], preferred_element_type=jnp.float32, precision=lax.Precision.HIGHEST)


def _tc_bn_mish_linear(h, g, be, wd, ws, b):
    return pl.pallas_call(
        _bn_mish_linear_body,
        out_shape=[
            jax.ShapeDtypeStruct((N, D), jnp.float32),
            jax.ShapeDtypeStruct((N, D), jnp.float32),
        ],
    )(h, g.reshape(1, D), be.reshape(1, D), wd, ws, b.reshape(1, D))


def _bn_mish_body(h_ref, g_ref, be_ref, o_ref):
    h = h_ref[...]
    mean = jnp.mean(h, axis=0, keepdims=True)
    var = jnp.mean((h - mean) ** 2, axis=0, keepdims=True)
    hn = (h - mean) * lax.rsqrt(var + 1e-5) * g_ref[...] + be_ref[...]
    o_ref[...] = _mish(hn)


def _tc_bn_mish(h, g, be):
    return pl.pallas_call(
        _bn_mish_body,
        out_shape=jax.ShapeDtypeStruct((N, D), jnp.float32),
    )(h, g.reshape(1, D), be.reshape(1, D))


def _bucket_body(dst_hbm, lst_hbm, cnt_hbm, dstbuf, pbuf, cbuf):
    wid = _wid()
    lo = wid * NB
    zero = jnp.full((16,), NB, jnp.int32)

    def zstep(i, carry):
        pbuf[pl.ds(i * 16, 16)] = zero
        return carry

    lax.fori_loop(0, (FLUSH + 144) // 16, zstep, 0)
    iota16 = lax.iota(jnp.int32, 16)

    CH = 3200

    def chunk(c, carry):
        pltpu.sync_copy(dst_hbm.at[pl.ds(pl.multiple_of(c * CH, 8), CH)], dstbuf)

        def step(i, carry2):
            cur, written = carry2
            base = c * CH + i * 128
            vs = [dstbuf[pl.ds(i * 128 + u * 16, 16)] for u in range(8)]
            ms = [(v >= lo) & (v < lo + NB) for v in vs]
            cs = [jnp.cumsum(m.astype(jnp.int32)) for m in ms]
            cnts = [cc[15] for cc in cs]
            packs = [(iota16 + (base + u * 16)) * 512 + (vs[u] - lo) for u in range(8)]
            for u in range(8):
                plsc.store_scatter(pbuf, [cur + cs[u] - 1], packs[u], mask=ms[u])
                cur = cur + cnts[u]
            do = cur >= FLUSH

            @pl.when(do)
            def _():
                pltpu.sync_copy(pbuf.at[pl.ds(0, FLUSH)], lst_hbm.at[pl.ds(pl.multiple_of(wid * CAP + written, 8), FLUSH)])
                for u in range(9):
                    pbuf[pl.ds(u * 16, 16)] = pbuf[pl.ds(FLUSH + u * 16, 16)]

            cur = lax.select(do, cur - FLUSH, cur)
            written = lax.select(do, written + FLUSH, written)
            return (cur, written)

        return lax.fori_loop(0, CH // 128, step, carry)

    cur, written = lax.fori_loop(0, E // CH, chunk, (0, 0))
    pltpu.sync_copy(pbuf.at[pl.ds(0, FLUSH)], lst_hbm.at[pl.ds(pl.multiple_of(wid * CAP + written, 8), FLUSH)])
    cbuf[pl.ds(0, 16)] = jnp.zeros((16,), jnp.int32) + (written + cur)
    pltpu.sync_copy(cbuf.at[pl.ds(0, 8)], cnt_hbm.at[pl.ds(pl.multiple_of(wid * 8, 8), 8)])


def _sc_bucket(dst):
    f = functools.partial(
        pl.kernel,
        out_type=[
            jax.ShapeDtypeStruct((NT * CAP,), jnp.int32),
            jax.ShapeDtypeStruct((NT * 8,), jnp.int32),
        ],
        mesh=_sc_mesh,
        compiler_params=pltpu.CompilerParams(needs_layout_passes=False),
        scratch_types=[
            pltpu.VMEM((3200,), jnp.int32),
            pltpu.VMEM((FLUSH + 144,), jnp.int32),
            pltpu.VMEM((16,), jnp.int32),
        ],
    )
    return f(_bucket_body)(dst)


def _smax_body(h_hbm, lst_hbm, cnt_hbm, out_hbm, acc, lbuf, idbuf, dlbuf, rows, cbuf, sem0, sem1):
    wid = _wid()
    neg = jnp.full((16,), NEG, jnp.float32)
    iota16 = lax.iota(jnp.int32, 16)
    sems = (sem0, sem1)

    def initr(r, carry):
        for k in range(8):
            acc[r, pl.ds(k * 16, 16)] = neg
        return carry

    lax.fori_loop(0, NB + 1, initr, 0)
    pltpu.sync_copy(cnt_hbm.at[pl.ds(pl.multiple_of(wid * 8, 8), 8)], cbuf.at[pl.ds(0, 8)])
    cnt = cbuf[pl.ds(0, 16)][0]
    nch = lax.div(cnt + (CHS - 1), CHS)

    def start(c, b):
        off = c * CHS
        pltpu.sync_copy(lst_hbm.at[pl.ds(pl.multiple_of(wid * CAP + off, 8), CHS)], lbuf.at[pl.ds(b * CHS, CHS)])
        for j in range(CHS // 16):
            lv = lbuf[pl.ds(b * CHS + j * 16, 16)]
            dl = lv & 511
            valid = (off + j * 16 + iota16) < cnt
            idbuf[pl.ds(b * CHS + j * 16, 16)] = lax.shift_right_logical(lv, 9)
            dlbuf[pl.ds(b * (CHS + 16) + j * 16, 16)] = jnp.where(valid, dl, NB)
        pltpu.async_copy(h_hbm.at[idbuf.at[pl.ds(b * CHS, CHS)]], rows.at[b], sems[b])

    def finish(b):
        pltpu.make_async_copy(h_hbm.at[idbuf.at[pl.ds(b * CHS, CHS)]], rows.at[b], sems[b]).wait()

        def edge(e4, carry2):
            e = e4 * 4
            lds = [dlbuf[pl.ds(b * (CHS + 16) + e + u, 16)][0] for u in range(4)]
            rs = [[rows[b, e + u, pl.ds(k * 16, 16)] for k in range(8)] for u in range(4)]
            for u in range(4):
                for k in range(8):
                    s = pl.ds(k * 16, 16)
                    acc[lds[u], s] = jnp.maximum(acc[lds[u], s], rs[u][k])
            return carry2

        lax.fori_loop(0, CHS // 4, edge, 0)

    @pl.when(0 < nch)
    def _():
        start(0, 0)

    @pl.when(1 < nch)
    def _():
        start(1, 1)

    def pair(p, carry):
        for b in range(2):
            c = 2 * p + b

            @pl.when(c < nch)
            def _():
                finish(b)

                @pl.when(c + 2 < nch)
                def _():
                    start(c + 2, b)

        return carry

    lax.fori_loop(0, lax.div(nch + 1, 2), pair, 0)

    def finr(r, carry):
        for k in range(8):
            s = pl.ds(k * 16, 16)
            v = acc[r, s]
            acc[r, s] = jnp.where(v == NEG, 0.0, v)
        return carry

    lax.fori_loop(0, NB, finr, 0)
    pltpu.sync_copy(acc.at[pl.ds(0, NB)], out_hbm.at[pl.ds(wid * NB, NB)])


def _sc_scatter_max(h, lst, cnts):
    f = functools.partial(
        pl.kernel,
        out_type=jax.ShapeDtypeStruct((NPAD, D), jnp.float32),
        mesh=_sc_mesh,
        compiler_params=pltpu.CompilerParams(needs_layout_passes=False),
        scratch_types=[
            pltpu.VMEM((NB + 1, D), jnp.float32),
            pltpu.VMEM((2 * CHS,), jnp.int32),
            pltpu.VMEM((2 * CHS,), jnp.int32),
            pltpu.VMEM((2 * (CHS + 16),), jnp.int32),
            pltpu.VMEM((2, CHS, D), jnp.float32),
            pltpu.VMEM((16,), jnp.int32),
            pltpu.SemaphoreType.DMA,
            pltpu.SemaphoreType.DMA,
        ],
    )
    return f(_smax_body)(h, lst, cnts)


def _gadd_body(a_hbm, b_hbm, src_hbm, dst_hbm, p_hbm, sbuf, dbuf, ra, rb, sem0, sem1):
    wid = _wid()
    nchunk = p_hbm.shape[0] // CHG
    sems = (sem0, sem1)
    HG = CHG // 2

    def gathers(b):
        yield a_hbm.at[dbuf.at[pl.ds(b * CHG, HG)]], ra.at[b, pl.ds(0, HG)]
        yield a_hbm.at[dbuf.at[pl.ds(b * CHG + HG, HG)]], ra.at[b, pl.ds(HG, HG)]
        yield b_hbm.at[sbuf.at[pl.ds(b * CHG, HG)]], rb.at[b, pl.ds(0, HG)]
        yield b_hbm.at[sbuf.at[pl.ds(b * CHG + HG, HG)]], rb.at[b, pl.ds(HG, HG)]

    def start(c, b):
        off = c * CHG
        pltpu.sync_copy(dst_hbm.at[pl.ds(pl.multiple_of(off, 8), CHG)], dbuf.at[pl.ds(b * CHG, CHG)])
        pltpu.sync_copy(src_hbm.at[pl.ds(pl.multiple_of(off, 8), CHG)], sbuf.at[pl.ds(b * CHG, CHG)])
        for s_ref, d_ref in gathers(b):
            pltpu.async_copy(s_ref, d_ref, sems[b])

    def finish(c, b):
        for s_ref, d_ref in gathers(b):
            pltpu.make_async_copy(s_ref, d_ref, sems[b]).wait()

        def addr(r2, carry2):
            r = r2 * 2
            for u in range(2):
                for k2 in range(8):
                    s = pl.ds(k2 * 16, 16)
                    ra[b, r + u, s] = ra[b, r + u, s] + rb[b, r + u, s]
            return carry2

        lax.fori_loop(0, CHG // 2, addr, 0)
        pltpu.sync_copy(ra.at[b], p_hbm.at[pl.ds(c * CHG, CHG)])

    c0 = wid
    c1 = wid + NT

    @pl.when(c0 < nchunk)
    def _():
        start(c0, 0)

    @pl.when(c1 < nchunk)
    def _():
        start(c1, 1)

    def it(k, carry):
        for b in range(2):
            c = wid + (2 * k + b) * NT

            @pl.when(c < nchunk)
            def _():
                finish(c, b)

                @pl.when(c + 2 * NT < nchunk)
                def _():
                    start(c + 2 * NT, b)

        return carry

    nit = (nchunk // NT + 2) // 2
    lax.fori_loop(0, nit, it, 0)


def _sc_gather_add(a, b, src, dst):
    ne = src.shape[0]
    f = functools.partial(
        pl.kernel,
        out_type=jax.ShapeDtypeStruct((ne, D), jnp.float32),
        mesh=_sc_mesh,
        compiler_params=pltpu.CompilerParams(needs_layout_passes=False),
        scratch_types=[
            pltpu.VMEM((2 * CHG,), jnp.int32),
            pltpu.VMEM((2 * CHG,), jnp.int32),
            pltpu.VMEM((2, CHG, D), jnp.float32),
            pltpu.VMEM((2, CHG, D), jnp.float32),
            pltpu.SemaphoreType.DMA,
            pltpu.SemaphoreType.DMA,
        ],
    )
    return f(_gadd_body)(a, b, src, dst)


def kernel(x, edge_index, edge_attr, W1a, b1a, W2a, b2a, g1, be1, W1b, b1b, W2b, b2b, g2, be2):
    src = edge_index[0]
    dst = edge_index[1]
    wd1 = W1a[:D] - W1a[D:]
    ws1 = W1a[D:]
    wd2 = W1b[:D] - W1b[D:]
    ws2 = W1b[D:]

    lst, cnts = _sc_bucket(dst)
    a1, b1 = _tc_node_linear(x, wd1, ws1, b1a)
    p1 = _sc_gather_add(a1, b1, src, dst)
    h1 = _tc_mlp(p1, W2a, b2a)
    m1 = _sc_scatter_max(h1, lst, cnts)[:N]
    a2, b2 = _tc_bn_mish_linear(m1, g1, be1, wd2, ws2, b1b)
    nparts = 5
    ep = E // nparts
    h2 = None
    for q in range(nparts):
        pq = _sc_gather_add(a2, b2, src[q * ep:(q + 1) * ep], dst[q * ep:(q + 1) * ep])
        h2 = _tc_mlp_part(pq, W2b, b2b, q, nparts, h2)
    m2 = _sc_scatter_max(h2, lst, cnts)[:N]
    out = _tc_bn_mish(m2, g2, be2)
    return (out, edge_index, edge_attr)
